# baseline (speedup 1.0000x reference)
import jax
import jax.numpy as jnp
from jax.experimental import pallas as pl

N_USERS = 30000
N_ITEMS = 20000
N = N_USERS + N_ITEMS
D = 64
MIN_NORM = 1e-6
EPS = 1e-7


def _matmul(x, w, block_m):
    M, K = x.shape
    _, Do = w.shape

    def mm_kernel(x_ref, w_ref, o_ref):
        o_ref[...] = jnp.dot(x_ref[...], w_ref[...],
                             preferred_element_type=jnp.float32)

    return pl.pallas_call(
        mm_kernel,
        grid=(M // block_m,),
        in_specs=[pl.BlockSpec((block_m, K), lambda i: (i, 0)),
                  pl.BlockSpec((K, Do), lambda i: (0, 0))],
        out_specs=pl.BlockSpec((block_m, Do), lambda i: (i, 0)),
        out_shape=jax.ShapeDtypeStruct((M, Do), jnp.float32),
    )(x, w)


def _proj_tan0(u):
    return jnp.concatenate([jnp.zeros_like(u[:, :1]), u[:, 1:]], axis=1)


def _hyp_proj(x, K):
    y = x[:, 1:]
    y_sqnorm = jnp.sum(y * y, axis=1, keepdims=True)
    first = jnp.maximum(jnp.sqrt(K + y_sqnorm), EPS)
    return jnp.concatenate([first, y], axis=1)


def _expmap0(u, c):
    K = 1.0 / c
    sqrtK = jnp.sqrt(K)
    x = u[:, 1:]
    xn = jnp.maximum(jnp.sqrt(jnp.sum(x * x, axis=1, keepdims=True)), MIN_NORM)
    theta = xn / sqrtK
    first = sqrtK * jnp.cosh(theta)
    rest = sqrtK * jnp.sinh(theta) * x / xn
    return _hyp_proj(jnp.concatenate([first, rest], axis=1), K)


def _logmap0(x, c):
    K = 1.0 / c
    sqrtK = jnp.sqrt(K)
    y = x[:, 1:]
    yn = jnp.maximum(jnp.sqrt(jnp.sum(y * y, axis=1, keepdims=True)), MIN_NORM)
    theta = jnp.maximum(x[:, :1] / sqrtK, 1.0 + EPS)
    ac = jnp.log(theta + jnp.sqrt(jnp.maximum(theta * theta - 1.0, 0.0)))
    return jnp.concatenate([jnp.zeros_like(x[:, :1]), sqrtK * ac * y / yn], axis=1)


def _spmm(h, t, v, emb):
    gathered = jnp.take(emb, t, axis=0) * v[:, None]
    return jax.ops.segment_sum(gathered, h, num_segments=N)


def kernel(user_ids, item_pos_ids, item_neg_ids, all_h_list, all_t_list,
           A_values, user_emb_id, item_emb_id, user_emb_img, user_emb_txt,
           image_feats, text_feats, Wi, bi, Wt, bt, c):
    ie_img = _matmul(image_feats, Wi, 400) + bi
    ie_txt = _matmul(text_feats, Wt, 400) + bt

    embs = []
    for ue, ie in ((user_emb_id, item_emb_id), (user_emb_img, ie_img),
                   (user_emb_txt, ie_txt)):
        ego = jnp.concatenate([ue, ie], axis=0)
        ego = _expmap0(_proj_tan0(ego), c)
        layers = []
        for _ in range(2):
            ego = _spmm(all_h_list, all_t_list, A_values, ego)
            layers.append(ego)
        allemb = jnp.mean(jnp.stack(layers, axis=1), axis=1)
        embs.append(_logmap0(allemb, c))
    e_id, e_img, e_txt = embs[0], 0.15 * embs[1], 0.75 * embs[2]

    def scores(e):
        u = jnp.take(e[:N_USERS], user_ids, axis=0)
        ip = jnp.take(e[N_USERS:], item_pos_ids, axis=0)
        ineg = jnp.take(e[N_USERS:], item_neg_ids, axis=0)
        return jnp.sum(u * ip, axis=1), jnp.sum(u * ineg, axis=1)

    ps_id, ns_id = scores(e_id)
    ps_im, ns_im = scores(e_img)
    ps_tx, ns_tx = scores(e_txt)
    pos = ps_id + ps_im + ps_tx
    neg = ns_id + ns_im + ns_tx
    return jnp.mean(-1.0 * jax.nn.log_sigmoid(pos - neg))


# speedup vs baseline: 6.0944x; 6.0944x over previous
import functools

import jax
import jax.numpy as jnp
from jax import lax
from jax.experimental import pallas as pl
from jax.experimental.pallas import tpu as pltpu
from jax.experimental.pallas import tpu_sc as plsc

N_USERS = 30000
N_ITEMS = 20000
N = N_USERS + N_ITEMS
E = 800000
D = 64
B = 4096
MIN_NORM = 1e-6
EPS = 1e-7

W = 32
NSLAB = 6
NTILE = 16
CHUNK = 256
SUB = CHUNK // 128
CPT = 200
CPT_ALLOC = 202
E_ALLOC = NTILE * CPT_ALLOC * CHUNK
ROWS_PER_TILE = N // NTILE
BT = 3 * B
BCH = BT // (NTILE * 128)


def _matmul_bias(x, w, b, block_m):
    M, K = x.shape
    _, Do = w.shape

    def mm_kernel(x_ref, w_ref, b_ref, o_ref):
        o_ref[...] = (jnp.dot(x_ref[...], w_ref[...],
                              preferred_element_type=jnp.float32)
                      + b_ref[...])

    return pl.pallas_call(
        mm_kernel,
        grid=(M // block_m,),
        in_specs=[pl.BlockSpec((block_m, K), lambda i: (i, 0)),
                  pl.BlockSpec((K, Do), lambda i: (0, 0)),
                  pl.BlockSpec((1, Do), lambda i: (0, 0))],
        out_specs=pl.BlockSpec((block_m, Do), lambda i: (i, 0)),
        out_shape=jax.ShapeDtypeStruct((M, Do), jnp.float32),
    )(x, w, b.reshape(1, Do))


def _expmap_slabs(ego3, c, block_r=1000):
    def k2(ego_ref, c_ref, o_ref):
        hf = pl.program_id(1)
        u = ego_ref[0]
        K = 1.0 / c_ref[0]
        sqrtK = jnp.sqrt(K)
        col = lax.broadcasted_iota(jnp.int32, u.shape, 1)
        x = jnp.where(col >= 1, u, 0.0)
        xn = jnp.maximum(jnp.sqrt(jnp.sum(x * x, axis=1, keepdims=True)),
                         MIN_NORM)
        theta = xn / sqrtK
        et = jnp.exp(theta)
        sinh_t = 0.5 * (et - 1.0 / et)
        rest = sqrtK * sinh_t * x / xn
        y_sqnorm = jnp.sum(rest * rest, axis=1, keepdims=True)
        first = jnp.maximum(jnp.sqrt(K + y_sqnorm), EPS)
        full = jnp.where(col >= 1, rest, first)
        o_ref[0] = jnp.where(hf == 0, full[:, :W], full[:, W:])

    return pl.pallas_call(
        k2,
        grid=(3, 2, N // block_r),
        in_specs=[pl.BlockSpec((1, block_r, D), lambda m, h, r: (m, r, 0)),
                  pl.BlockSpec(memory_space=pltpu.SMEM)],
        out_specs=pl.BlockSpec((1, block_r, W),
                               lambda m, h, r: (2 * m + h, r, 0)),
        out_shape=jax.ShapeDtypeStruct((NSLAB, N, W), jnp.float32),
    )(ego3, c.reshape(1))


def _sc_spmm(X, th, v, idx3):
    mesh = plsc.VectorSubcoreMesh(core_axis_name="c", subcore_axis_name="s")
    fdt = jnp.float32
    out_types = (jax.ShapeDtypeStruct((NSLAB, N, W), fdt),
                 jax.ShapeDtypeStruct((NSLAB, N, W), fdt),
                 jax.ShapeDtypeStruct((NSLAB, BT, W), fdt),
                 jax.ShapeDtypeStruct((NSLAB, BT, W), fdt))

    scratch = [
        pltpu.VMEM_SHARED((N, W), fdt),
        pltpu.VMEM((55, W), fdt),
        pltpu.VMEM((2 * SUB, 128), jnp.int32),
        pltpu.VMEM((2 * SUB, 128), jnp.int32),
        pltpu.VMEM((CHUNK,), fdt),
        pltpu.VMEM((CHUNK,), fdt),
        pltpu.VMEM((CHUNK, W), fdt),
        pltpu.VMEM((CHUNK, W), fdt),
        pltpu.VMEM((BCH, 128), jnp.int32),
        pltpu.SemaphoreType.DMA,
        pltpu.SemaphoreType.DMA,
        pltpu.SemaphoreType.DMA,
        pltpu.SemaphoreType.DMA,
        pltpu.SemaphoreType.DMA,
        pltpu.SemaphoreType.DMA,
    ]

    @functools.partial(pl.kernel, out_type=out_types, mesh=mesh,
                       scratch_types=scratch,
                       compiler_params=pltpu.CompilerParams(
                           use_tc_tiling_on_sc=False))
    def k3(X_hbm, th_hbm, v_hbm, idx_hbm, L1, L2, G1, G2,
           shared, zbuf, thA, thB, vA, vB, rowsA, rowsB, ibuf,
           stsA, stsB, gsA, gsB, ssA, ssB):
        core = lax.axis_index("c")
        tid = lax.axis_index("s")

        @pl.loop(0, 55, unroll=8)
        def _(i):
            zero16 = jnp.zeros((16,), fdt)
            zbuf[i, pl.ds(0, 16)] = zero16
            zbuf[i, pl.ds(16, 16)] = zero16

        share_off = tid * 3128

        pltpu.sync_copy(idx_hbm.at[pl.ds(BCH * tid, BCH)], ibuf)

        def stage(th_buf, v_buf, ck, sts):
            rbase = (tid * CPT_ALLOC + ck) * 2 * SUB
            vbase = (tid * CPT_ALLOC + ck) * CHUNK
            c1 = pltpu.async_copy(th_hbm.at[pl.ds(rbase, 2 * SUB)], th_buf, sts)
            c2 = pltpu.async_copy(v_hbm.at[pl.ds(vbase, CHUNK)], v_buf, sts)
            c1.wait()
            c2.wait()

        def gather_start(tbl, th_buf, rows, gs):
            for jj in range(SUB):
                pltpu.async_copy(tbl.at[th_buf.at[jj]],
                                 rows.at[pl.ds(jj * 128, 128)], gs)

        def gather_wait(tbl, th_buf, rows, gs):
            for jj in range(SUB):
                pltpu.make_async_copy(tbl.at[th_buf.at[jj]],
                                      rows.at[pl.ds(jj * 128, 128)], gs).wait()

        def scatter(th_buf, rows, ss):
            cps = [pltpu.async_copy(rows.at[pl.ds(jj * 128, 128)],
                                    shared.at[th_buf.at[SUB + jj]], ss,
                                    add=True)
                   for jj in range(SUB)]
            return cps

        def scale(rows, v_buf):
            @pl.loop(0, CHUNK, step=16)
            def _(k0):
                v16 = v_buf[pl.ds(k0, 16)]
                for i in range(16):
                    sv = lax.broadcast(v16[i], (16,))
                    k = k0 + i
                    rows[k, pl.ds(0, 16)] = rows[k, pl.ds(0, 16)] * sv
                    rows[k, pl.ds(16, 16)] = rows[k, pl.ds(16, 16)] * sv

        for j in range(3):
            s = 3 * core + j
            for layer in range(2):
                table = (X_hbm if layer == 0 else L1).at[s]
                Ldst = (L1 if layer == 0 else L2).at[s]

                @pl.loop(0, 56)
                def _(i):
                    pltpu.sync_copy(
                        zbuf, shared.at[pl.ds(share_off + i * 55, 55)])

                @pl.when(tid < 15)
                def _():
                    pltpu.sync_copy(zbuf.at[pl.ds(0, 48)],
                                    shared.at[pl.ds(share_off + 3080, 48)])
                plsc.subcore_barrier()

                stage(thA, vA, 0, stsA)
                gather_start(table, thA, rowsA, gsA)
                stage(thB, vB, 1, stsB)
                gather_start(table, thB, rowsB, gsB)

                @pl.loop(0, CPT // 2)
                def _(g):
                    c0 = 2 * g
                    gather_wait(table, thA, rowsA, gsA)
                    scale(rowsA, vA)
                    scA = scatter(thA, rowsA, ssA)
                    gather_wait(table, thB, rowsB, gsB)
                    scale(rowsB, vB)
                    scB = scatter(thB, rowsB, ssB)
                    for cp in scA:
                        cp.wait()
                    stage(thA, vA, c0 + 2, stsA)
                    gather_start(table, thA, rowsA, gsA)
                    for cp in scB:
                        cp.wait()
                    stage(thB, vB, c0 + 3, stsB)
                    gather_start(table, thB, rowsB, gsB)

                gather_wait(table, thA, rowsA, gsA)
                gather_wait(table, thB, rowsB, gsB)
                plsc.subcore_barrier()

                pltpu.sync_copy(shared.at[pl.ds(share_off, 3080)],
                                Ldst.at[pl.ds(share_off, 3080)])

                @pl.when(tid < 15)
                def _():
                    pltpu.sync_copy(shared.at[pl.ds(share_off + 3080, 48)],
                                    Ldst.at[pl.ds(share_off + 3080, 48)])
                plsc.subcore_barrier()

            for b in range(BCH // 2):
                cps = []
                for jj in range(2):
                    cps.append(pltpu.async_copy(
                        L1.at[s].at[ibuf.at[2 * b + jj]],
                        rowsA.at[pl.ds(jj * 128, 128)], gsA))
                    cps.append(pltpu.async_copy(
                        L2.at[s].at[ibuf.at[2 * b + jj]],
                        rowsB.at[pl.ds(jj * 128, 128)], gsB))
                for cp in cps:
                    cp.wait()
                off = BCH * 128 * tid + 256 * b
                pltpu.sync_copy(rowsA, G1.at[s].at[pl.ds(off, 256)])
                pltpu.sync_copy(rowsB, G2.at[s].at[pl.ds(off, 256)])

    return k3(X, th, v, idx3)


def _loss(G1, G2, c, rb=512):
    nsteps = B // rb

    def k4(g1u, g1p, g1n, g2u, g2p, g2n, c_ref, o_ref):
        i = pl.program_id(0)
        K = 1.0 / c_ref[0]
        sqrtK = jnp.sqrt(K)

        def emap(g1b, g2b, m, wgt):
            a0 = (g1b[2 * m] + g2b[2 * m]) * 0.5
            a1 = (g1b[2 * m + 1] + g2b[2 * m + 1]) * 0.5
            x0 = a0[:, 0:1]
            sq = (jnp.sum(a0 * a0, axis=1, keepdims=True)
                  + jnp.sum(a1 * a1, axis=1, keepdims=True) - x0 * x0)
            yn = jnp.maximum(jnp.sqrt(jnp.maximum(sq, 0.0)), MIN_NORM)
            theta = jnp.maximum(x0 / sqrtK, 1.0 + EPS)
            ac = jnp.log(theta + jnp.sqrt(jnp.maximum(theta * theta - 1.0,
                                                      0.0)))
            f = wgt * sqrtK * ac / yn
            col = lax.broadcasted_iota(jnp.int32, a0.shape, 1)
            e0 = jnp.where(col >= 1, a0 * f, 0.0)
            e1 = a1 * f
            return e0, e1

        pos = jnp.zeros((rb, 1), jnp.float32)
        neg = jnp.zeros((rb, 1), jnp.float32)
        for m, wgt in ((0, 1.0), (1, 0.15), (2, 0.75)):
            eu0, eu1 = emap(g1u[...], g2u[...], m, wgt)
            ep0, ep1 = emap(g1p[...], g2p[...], m, wgt)
            en0, en1 = emap(g1n[...], g2n[...], m, wgt)
            pos = pos + (jnp.sum(eu0 * ep0, axis=1, keepdims=True)
                         + jnp.sum(eu1 * ep1, axis=1, keepdims=True))
            neg = neg + (jnp.sum(eu0 * en0, axis=1, keepdims=True)
                         + jnp.sum(eu1 * en1, axis=1, keepdims=True))
        d = pos - neg
        lossv = jnp.maximum(-d, 0.0) + jnp.log1p(jnp.exp(-jnp.abs(d)))
        partial = jnp.sum(lossv, axis=(0, 1), keepdims=True) / B

        @pl.when(i == 0)
        def _():
            o_ref[...] = jnp.zeros((1, 1), jnp.float32)

        o_ref[...] += partial

    blk = rb // 128
    ublock = pl.BlockSpec((NSLAB, rb, W), lambda i: (0, i, 0))
    pblock = pl.BlockSpec((NSLAB, rb, W), lambda i: (0, i + B // rb, 0))
    nblock = pl.BlockSpec((NSLAB, rb, W), lambda i: (0, i + 2 * B // rb, 0))
    return pl.pallas_call(
        k4,
        grid=(nsteps,),
        in_specs=[ublock, pblock, nblock, ublock, pblock, nblock,
                  pl.BlockSpec(memory_space=pltpu.SMEM)],
        out_specs=pl.BlockSpec((1, 1), lambda i: (0, 0)),
        out_shape=jax.ShapeDtypeStruct((1, 1), jnp.float32),
    )(G1, G1, G1, G2, G2, G2, c.reshape(1))


def _pack_edges(h, t, v):
    cap = NTILE * CPT * CHUNK
    pad = cap - E
    fill = (jnp.arange(pad, dtype=jnp.int32) * 61) % N
    t_p = jnp.concatenate([t.astype(jnp.int32), fill])
    h_p = jnp.concatenate([h.astype(jnp.int32), fill])
    v_p = jnp.concatenate([v, jnp.zeros((pad,), jnp.float32)])
    t_r = t_p.reshape(NTILE, CPT, SUB, 128)
    h_r = h_p.reshape(NTILE, CPT, SUB, 128)
    v_r = v_p.reshape(NTILE, CPT, CHUNK)
    zi = jnp.zeros((NTILE, 2, SUB, 128), jnp.int32)
    th = jnp.concatenate([t_r, h_r], axis=2)
    th = jnp.concatenate([th, jnp.concatenate([zi, zi], axis=2)], axis=1)
    v_r = jnp.concatenate([v_r, jnp.zeros((NTILE, 2, CHUNK), jnp.float32)],
                          axis=1)
    return (th.reshape(NTILE * CPT_ALLOC * 2 * SUB, 128),
            v_r.reshape(E_ALLOC))


def kernel(user_ids, item_pos_ids, item_neg_ids, all_h_list, all_t_list,
           A_values, user_emb_id, item_emb_id, user_emb_img, user_emb_txt,
           image_feats, text_feats, Wi, bi, Wt, bt, c):
    ie_img = _matmul_bias(image_feats, Wi, bi, 400)
    ie_txt = _matmul_bias(text_feats, Wt, bt, 400)
    ego3 = jnp.stack([
        jnp.concatenate([user_emb_id, item_emb_id], axis=0),
        jnp.concatenate([user_emb_img, ie_img], axis=0),
        jnp.concatenate([user_emb_txt, ie_txt], axis=0)])
    X = _expmap_slabs(ego3, c)

    th, v = _pack_edges(all_h_list, all_t_list, A_values)
    idx_all = jnp.concatenate([user_ids.astype(jnp.int32),
                               N_USERS + item_pos_ids.astype(jnp.int32),
                               N_USERS + item_neg_ids.astype(jnp.int32)])
    idx3 = idx_all.reshape(BT // 128, 128)

    _, _, G1, G2 = _sc_spmm(X, th, v, idx3)
    out = _loss(G1, G2, c)
    return out.reshape(())


# speedup vs baseline: 7.3007x; 1.1979x over previous
import functools

import jax
import jax.numpy as jnp
from jax import lax
from jax.experimental import pallas as pl
from jax.experimental.pallas import tpu as pltpu
from jax.experimental.pallas import tpu_sc as plsc

N_USERS = 30000
N_ITEMS = 20000
N = N_USERS + N_ITEMS
E = 800000
D = 64
B = 4096
MIN_NORM = 1e-6
EPS = 1e-7

W = 32
NSLAB = 6
NTILE = 16
CHUNK = 256
SUB = CHUNK // 128
CPT = 200
CPT_ALLOC = 202
E_ALLOC = NTILE * CPT_ALLOC * CHUNK
ROWS_PER_TILE = N // NTILE
BT = 3 * B
BCH = BT // (NTILE * 128)


def _matmul_bias(x, w, b, block_m):
    M, K = x.shape
    _, Do = w.shape

    def mm_kernel(x_ref, w_ref, b_ref, o_ref):
        o_ref[...] = (jnp.dot(x_ref[...], w_ref[...],
                              preferred_element_type=jnp.float32)
                      + b_ref[...])

    return pl.pallas_call(
        mm_kernel,
        grid=(M // block_m,),
        in_specs=[pl.BlockSpec((block_m, K), lambda i: (i, 0)),
                  pl.BlockSpec((K, Do), lambda i: (0, 0)),
                  pl.BlockSpec((1, Do), lambda i: (0, 0))],
        out_specs=pl.BlockSpec((block_m, Do), lambda i: (i, 0)),
        out_shape=jax.ShapeDtypeStruct((M, Do), jnp.float32),
    )(x, w, b.reshape(1, Do))


def _expmap_slabs(ego3, c, block_r=1000):
    def k2(ego_ref, c_ref, o_ref):
        hf = pl.program_id(1)
        u = ego_ref[0]
        K = 1.0 / c_ref[0]
        sqrtK = jnp.sqrt(K)
        col = lax.broadcasted_iota(jnp.int32, u.shape, 1)
        x = jnp.where(col >= 1, u, 0.0)
        xn = jnp.maximum(jnp.sqrt(jnp.sum(x * x, axis=1, keepdims=True)),
                         MIN_NORM)
        theta = xn / sqrtK
        et = jnp.exp(theta)
        sinh_t = 0.5 * (et - 1.0 / et)
        rest = sqrtK * sinh_t * x / xn
        y_sqnorm = jnp.sum(rest * rest, axis=1, keepdims=True)
        first = jnp.maximum(jnp.sqrt(K + y_sqnorm), EPS)
        full = jnp.where(col >= 1, rest, first)
        o_ref[0] = jnp.where(hf == 0, full[:, :W], full[:, W:])

    return pl.pallas_call(
        k2,
        grid=(3, 2, N // block_r),
        in_specs=[pl.BlockSpec((1, block_r, D), lambda m, h, r: (m, r, 0)),
                  pl.BlockSpec(memory_space=pltpu.SMEM)],
        out_specs=pl.BlockSpec((1, block_r, W),
                               lambda m, h, r: (2 * m + h, r, 0)),
        out_shape=jax.ShapeDtypeStruct((NSLAB, N, W), jnp.float32),
    )(ego3, c.reshape(1))


def _sc_spmm(X, t1, h1, v1, idx3):
    mesh = plsc.VectorSubcoreMesh(core_axis_name="c", subcore_axis_name="s")
    fdt = jnp.float32
    out_types = (jax.ShapeDtypeStruct((NSLAB, N, W), fdt),
                 jax.ShapeDtypeStruct((NSLAB, N, W), fdt),
                 jax.ShapeDtypeStruct((NSLAB, BT, W), fdt),
                 jax.ShapeDtypeStruct((NSLAB, BT, W), fdt))

    scratch = [
        pltpu.VMEM_SHARED((N, W), fdt),
        pltpu.VMEM((55, W), fdt),
        pltpu.VMEM((CHUNK,), jnp.int32),
        pltpu.VMEM((CHUNK,), jnp.int32),
        pltpu.VMEM((CHUNK,), jnp.int32),
        pltpu.VMEM((CHUNK,), jnp.int32),
        pltpu.VMEM((CHUNK,), fdt),
        pltpu.VMEM((CHUNK,), fdt),
        pltpu.VMEM((CHUNK, W), fdt),
        pltpu.VMEM((CHUNK, W), fdt),
        pltpu.VMEM((BCH, 128), jnp.int32),
        pltpu.SemaphoreType.DMA,
        pltpu.SemaphoreType.DMA,
        pltpu.SemaphoreType.DMA,
        pltpu.SemaphoreType.DMA,
        pltpu.SemaphoreType.DMA,
        pltpu.SemaphoreType.DMA,
    ]

    @functools.partial(pl.kernel, out_type=out_types, mesh=mesh,
                       scratch_types=scratch,
                       compiler_params=pltpu.CompilerParams(
                           use_tc_tiling_on_sc=False))
    def k3(X_hbm, t_hbm, h_hbm, v_hbm, idx_hbm, L1, L2, G1, G2,
           shared, zbuf, tA, tB, hA, hB, vA, vB, rowsA, rowsB, ibuf,
           stsA, stsB, gsA, gsB, ssA, ssB):
        core = lax.axis_index("c")
        tid = lax.axis_index("s")

        @pl.loop(0, 55, unroll=8)
        def _(i):
            zero16 = jnp.zeros((16,), fdt)
            zbuf[i, pl.ds(0, 16)] = zero16
            zbuf[i, pl.ds(16, 16)] = zero16

        share_off = tid * 3128

        pltpu.sync_copy(idx_hbm.at[pl.ds(BCH * tid, BCH)], ibuf)

        def stage(t_buf, h_buf, v_buf, ck, sts):
            base = (tid * CPT_ALLOC + ck) * CHUNK
            cps = (pltpu.async_copy(t_hbm.at[pl.ds(base, CHUNK)], t_buf, sts),
                   pltpu.async_copy(h_hbm.at[pl.ds(base, CHUNK)], h_buf, sts),
                   pltpu.async_copy(v_hbm.at[pl.ds(base, CHUNK)], v_buf, sts))
            for cp in cps:
                cp.wait()

        def gather_start(tbl, t_buf, rows, gs):
            pltpu.async_copy(tbl.at[t_buf], rows, gs)

        def gather_wait(tbl, t_buf, rows, gs):
            pltpu.make_async_copy(tbl.at[t_buf], rows, gs).wait()

        def scatter(h_buf, rows, ss):
            return [pltpu.async_copy(rows, shared.at[h_buf], ss, add=True)]

        def scale(rows, v_buf):
            @pl.loop(0, CHUNK, step=16)
            def _(k0):
                v16 = v_buf[pl.ds(k0, 16)]
                for i in range(16):
                    sv = lax.broadcast(v16[i], (16,))
                    k = k0 + i
                    rows[k, pl.ds(0, 16)] = rows[k, pl.ds(0, 16)] * sv
                    rows[k, pl.ds(16, 16)] = rows[k, pl.ds(16, 16)] * sv

        for j in range(3):
            s = 3 * core + j
            for layer in range(2):
                table = (X_hbm if layer == 0 else L1).at[s]
                Ldst = (L1 if layer == 0 else L2).at[s]

                @pl.loop(0, 56)
                def _(i):
                    pltpu.sync_copy(
                        zbuf, shared.at[pl.ds(share_off + i * 55, 55)])

                @pl.when(tid < 15)
                def _():
                    pltpu.sync_copy(zbuf.at[pl.ds(0, 48)],
                                    shared.at[pl.ds(share_off + 3080, 48)])
                plsc.subcore_barrier()

                stage(tA, hA, vA, 0, stsA)
                gather_start(table, tA, rowsA, gsA)
                stage(tB, hB, vB, 1, stsB)
                gather_start(table, tB, rowsB, gsB)

                @pl.loop(0, CPT // 2)
                def _(g):
                    c0 = 2 * g
                    gather_wait(table, tA, rowsA, gsA)
                    scale(rowsA, vA)
                    scA = scatter(hA, rowsA, ssA)
                    gather_wait(table, tB, rowsB, gsB)
                    scale(rowsB, vB)
                    scB = scatter(hB, rowsB, ssB)
                    for cp in scA:
                        cp.wait()
                    stage(tA, hA, vA, c0 + 2, stsA)
                    gather_start(table, tA, rowsA, gsA)
                    for cp in scB:
                        cp.wait()
                    stage(tB, hB, vB, c0 + 3, stsB)
                    gather_start(table, tB, rowsB, gsB)

                gather_wait(table, tA, rowsA, gsA)
                gather_wait(table, tB, rowsB, gsB)
                plsc.subcore_barrier()

                pltpu.sync_copy(shared.at[pl.ds(share_off, 3080)],
                                Ldst.at[pl.ds(share_off, 3080)])

                @pl.when(tid < 15)
                def _():
                    pltpu.sync_copy(shared.at[pl.ds(share_off + 3080, 48)],
                                    Ldst.at[pl.ds(share_off + 3080, 48)])
                plsc.subcore_barrier()

            for b in range(BCH // 2):
                cps = []
                for jj in range(2):
                    cps.append(pltpu.async_copy(
                        L1.at[s].at[ibuf.at[2 * b + jj]],
                        rowsA.at[pl.ds(jj * 128, 128)], gsA))
                    cps.append(pltpu.async_copy(
                        L2.at[s].at[ibuf.at[2 * b + jj]],
                        rowsB.at[pl.ds(jj * 128, 128)], gsB))
                for cp in cps:
                    cp.wait()
                off = BCH * 128 * tid + 256 * b
                pltpu.sync_copy(rowsA, G1.at[s].at[pl.ds(off, 256)])
                pltpu.sync_copy(rowsB, G2.at[s].at[pl.ds(off, 256)])

    return k3(X, t1, h1, v1, idx3)


def _loss(G1, G2, c, rb=512):
    nsteps = B // rb

    def k4(g1u, g1p, g1n, g2u, g2p, g2n, c_ref, o_ref):
        i = pl.program_id(0)
        K = 1.0 / c_ref[0]
        sqrtK = jnp.sqrt(K)

        def emap(g1b, g2b, m, wgt):
            a0 = (g1b[2 * m] + g2b[2 * m]) * 0.5
            a1 = (g1b[2 * m + 1] + g2b[2 * m + 1]) * 0.5
            x0 = a0[:, 0:1]
            sq = (jnp.sum(a0 * a0, axis=1, keepdims=True)
                  + jnp.sum(a1 * a1, axis=1, keepdims=True) - x0 * x0)
            yn = jnp.maximum(jnp.sqrt(jnp.maximum(sq, 0.0)), MIN_NORM)
            theta = jnp.maximum(x0 / sqrtK, 1.0 + EPS)
            ac = jnp.log(theta + jnp.sqrt(jnp.maximum(theta * theta - 1.0,
                                                      0.0)))
            f = wgt * sqrtK * ac / yn
            col = lax.broadcasted_iota(jnp.int32, a0.shape, 1)
            e0 = jnp.where(col >= 1, a0 * f, 0.0)
            e1 = a1 * f
            return e0, e1

        pos = jnp.zeros((rb, 1), jnp.float32)
        neg = jnp.zeros((rb, 1), jnp.float32)
        for m, wgt in ((0, 1.0), (1, 0.15), (2, 0.75)):
            eu0, eu1 = emap(g1u[...], g2u[...], m, wgt)
            ep0, ep1 = emap(g1p[...], g2p[...], m, wgt)
            en0, en1 = emap(g1n[...], g2n[...], m, wgt)
            pos = pos + (jnp.sum(eu0 * ep0, axis=1, keepdims=True)
                         + jnp.sum(eu1 * ep1, axis=1, keepdims=True))
            neg = neg + (jnp.sum(eu0 * en0, axis=1, keepdims=True)
                         + jnp.sum(eu1 * en1, axis=1, keepdims=True))
        d = pos - neg
        lossv = jnp.maximum(-d, 0.0) + jnp.log1p(jnp.exp(-jnp.abs(d)))
        partial = jnp.sum(lossv, axis=(0, 1), keepdims=True) / B

        @pl.when(i == 0)
        def _():
            o_ref[...] = jnp.zeros((1, 1), jnp.float32)

        o_ref[...] += partial

    blk = rb // 128
    ublock = pl.BlockSpec((NSLAB, rb, W), lambda i: (0, i, 0))
    pblock = pl.BlockSpec((NSLAB, rb, W), lambda i: (0, i + B // rb, 0))
    nblock = pl.BlockSpec((NSLAB, rb, W), lambda i: (0, i + 2 * B // rb, 0))
    return pl.pallas_call(
        k4,
        grid=(nsteps,),
        in_specs=[ublock, pblock, nblock, ublock, pblock, nblock,
                  pl.BlockSpec(memory_space=pltpu.SMEM)],
        out_specs=pl.BlockSpec((1, 1), lambda i: (0, 0)),
        out_shape=jax.ShapeDtypeStruct((1, 1), jnp.float32),
    )(G1, G1, G1, G2, G2, G2, c.reshape(1))


def _pack_edges(h, t, v):
    cap = NTILE * CPT * CHUNK
    pad = cap - E
    fill = (jnp.arange(pad, dtype=jnp.int32) * 61) % N
    fill2 = (jnp.arange(NTILE * 2 * CHUNK, dtype=jnp.int32) * 61) % N

    def lay(x, filler):
        x_r = jnp.concatenate([x, filler]).reshape(NTILE, CPT, CHUNK)
        x_r = jnp.concatenate(
            [x_r, fill2.reshape(NTILE, 2, CHUNK).astype(x.dtype)], axis=1)
        return x_r.reshape(E_ALLOC)

    t1 = lay(t.astype(jnp.int32), fill)
    h1 = lay(h.astype(jnp.int32), fill)
    v1 = lay(v, jnp.zeros((pad,), jnp.float32))
    return t1, h1, v1


def kernel(user_ids, item_pos_ids, item_neg_ids, all_h_list, all_t_list,
           A_values, user_emb_id, item_emb_id, user_emb_img, user_emb_txt,
           image_feats, text_feats, Wi, bi, Wt, bt, c):
    ie_img = _matmul_bias(image_feats, Wi, bi, 400)
    ie_txt = _matmul_bias(text_feats, Wt, bt, 400)
    ego3 = jnp.stack([
        jnp.concatenate([user_emb_id, item_emb_id], axis=0),
        jnp.concatenate([user_emb_img, ie_img], axis=0),
        jnp.concatenate([user_emb_txt, ie_txt], axis=0)])
    X = _expmap_slabs(ego3, c)

    t1, h1, v1 = _pack_edges(all_h_list, all_t_list, A_values)
    idx_all = jnp.concatenate([user_ids.astype(jnp.int32),
                               N_USERS + item_pos_ids.astype(jnp.int32),
                               N_USERS + item_neg_ids.astype(jnp.int32)])
    idx3 = idx_all.reshape(BT // 128, 128)

    _, _, G1, G2 = _sc_spmm(X, t1, h1, v1, idx3)
    out = _loss(G1, G2, c)
    return out.reshape(())


# speedup vs baseline: 7.9697x; 1.0916x over previous
import functools

import jax
import jax.numpy as jnp
from jax import lax
from jax.experimental import pallas as pl
from jax.experimental.pallas import tpu as pltpu
from jax.experimental.pallas import tpu_sc as plsc

N_USERS = 30000
N_ITEMS = 20000
N = N_USERS + N_ITEMS
E = 800000
D = 64
B = 4096
MIN_NORM = 1e-6
EPS = 1e-7

W = 32
NSLAB = 6
NTILE = 16
CHUNK = 256
SUB = CHUNK // 128
CPT = 201
CPT_ALLOC = 204
E_ALLOC = NTILE * CPT_ALLOC * CHUNK
ROWS_PER_TILE = N // NTILE
BT = 3 * B
BCH = BT // (NTILE * 128)


def _matmul_bias(x, w, b, block_m):
    M, K = x.shape
    _, Do = w.shape

    def mm_kernel(x_ref, w_ref, b_ref, o_ref):
        o_ref[...] = (jnp.dot(x_ref[...], w_ref[...],
                              preferred_element_type=jnp.float32)
                      + b_ref[...])

    return pl.pallas_call(
        mm_kernel,
        grid=(M // block_m,),
        in_specs=[pl.BlockSpec((block_m, K), lambda i: (i, 0)),
                  pl.BlockSpec((K, Do), lambda i: (0, 0)),
                  pl.BlockSpec((1, Do), lambda i: (0, 0))],
        out_specs=pl.BlockSpec((block_m, Do), lambda i: (i, 0)),
        out_shape=jax.ShapeDtypeStruct((M, Do), jnp.float32),
    )(x, w, b.reshape(1, Do))


def _expmap_slabs(ego3, c, block_r=1000):
    def k2(ego_ref, c_ref, o_ref):
        hf = pl.program_id(1)
        u = ego_ref[0]
        K = 1.0 / c_ref[0]
        sqrtK = jnp.sqrt(K)
        col = lax.broadcasted_iota(jnp.int32, u.shape, 1)
        x = jnp.where(col >= 1, u, 0.0)
        xn = jnp.maximum(jnp.sqrt(jnp.sum(x * x, axis=1, keepdims=True)),
                         MIN_NORM)
        theta = xn / sqrtK
        et = jnp.exp(theta)
        sinh_t = 0.5 * (et - 1.0 / et)
        rest = sqrtK * sinh_t * x / xn
        y_sqnorm = jnp.sum(rest * rest, axis=1, keepdims=True)
        first = jnp.maximum(jnp.sqrt(K + y_sqnorm), EPS)
        full = jnp.where(col >= 1, rest, first)
        o_ref[0] = jnp.where(hf == 0, full[:, :W], full[:, W:])

    return pl.pallas_call(
        k2,
        grid=(3, 2, N // block_r),
        in_specs=[pl.BlockSpec((1, block_r, D), lambda m, h, r: (m, r, 0)),
                  pl.BlockSpec(memory_space=pltpu.SMEM)],
        out_specs=pl.BlockSpec((1, block_r, W),
                               lambda m, h, r: (2 * m + h, r, 0)),
        out_shape=jax.ShapeDtypeStruct((NSLAB, N, W), jnp.float32),
    )(ego3, c.reshape(1))


def _sc_spmm(X, tv, h1, idx3):
    mesh = plsc.VectorSubcoreMesh(core_axis_name="c", subcore_axis_name="s")
    fdt = jnp.float32
    out_types = (jax.ShapeDtypeStruct((NSLAB, N, W), fdt),
                 jax.ShapeDtypeStruct((NSLAB, N, W), fdt),
                 jax.ShapeDtypeStruct((NSLAB, BT, W), fdt),
                 jax.ShapeDtypeStruct((NSLAB, BT, W), fdt))

    scratch = [
        pltpu.VMEM_SHARED((N, W), fdt),
        pltpu.VMEM((55, W), fdt),
        pltpu.VMEM((2 * CHUNK,), jnp.int32),
        pltpu.VMEM((2 * CHUNK,), jnp.int32),
        pltpu.VMEM((2 * CHUNK,), jnp.int32),
        pltpu.VMEM((CHUNK,), jnp.int32),
        pltpu.VMEM((CHUNK,), jnp.int32),
        pltpu.VMEM((CHUNK,), jnp.int32),
        pltpu.VMEM((CHUNK, W), fdt),
        pltpu.VMEM((CHUNK, W), fdt),
        pltpu.VMEM((CHUNK, W), fdt),
        pltpu.VMEM((BCH, 128), jnp.int32),
        pltpu.SemaphoreType.DMA,
        pltpu.SemaphoreType.DMA,
        pltpu.SemaphoreType.DMA,
        pltpu.SemaphoreType.DMA,
        pltpu.SemaphoreType.DMA,
        pltpu.SemaphoreType.DMA,
        pltpu.SemaphoreType.DMA,
        pltpu.SemaphoreType.DMA,
        pltpu.SemaphoreType.DMA,
    ]

    @functools.partial(pl.kernel, out_type=out_types, mesh=mesh,
                       scratch_types=scratch,
                       compiler_params=pltpu.CompilerParams(
                           use_tc_tiling_on_sc=False,
                           needs_layout_passes=False))
    def k3(X_hbm, tv_hbm, h_hbm, idx_hbm, L1, L2, G1, G2,
           shared, zbuf, tvA, tvB, tvC, hA, hB, hC, rowsA, rowsB, rowsC, ibuf,
           stsA, stsB, stsC, gsA, gsB, gsC, ssA, ssB, ssC):
        core = lax.axis_index("c")
        tid = lax.axis_index("s")

        @pl.loop(0, 55, unroll=8)
        def _(i):
            zero16 = jnp.zeros((16,), fdt)
            zbuf[i, pl.ds(0, 16)] = zero16
            zbuf[i, pl.ds(16, 16)] = zero16

        share_off = tid * 3128

        pltpu.sync_copy(idx_hbm.at[pl.ds(BCH * tid, BCH)], ibuf)

        def stage(tv_buf, h_buf, ck, sts):
            base = (tid * CPT_ALLOC + ck) * CHUNK
            cps = (pltpu.async_copy(tv_hbm.at[pl.ds(2 * base, 2 * CHUNK)],
                                    tv_buf, sts),
                   pltpu.async_copy(h_hbm.at[pl.ds(base, CHUNK)], h_buf, sts))
            for cp in cps:
                cp.wait()

        def gather_start(tbl, tv_buf, rows, gs):
            pltpu.async_copy(tbl.at[tv_buf.at[pl.ds(0, CHUNK)]], rows, gs)

        def gather_wait(tbl, tv_buf, rows, gs):
            pltpu.make_async_copy(tbl.at[tv_buf.at[pl.ds(0, CHUNK)]],
                                  rows, gs).wait()

        def scatter(h_buf, rows, ss):
            return [pltpu.async_copy(rows, shared.at[h_buf], ss, add=True)]

        def scale(rows, tv_buf):
            @pl.loop(0, CHUNK, step=16)
            def _(k0):
                v16 = plsc.bitcast(tv_buf[pl.ds(CHUNK + k0, 16)], fdt)
                for i in range(16):
                    sv = lax.broadcast(v16[i], (16,))
                    k = k0 + i
                    rows[k, pl.ds(0, 16)] = rows[k, pl.ds(0, 16)] * sv
                    rows[k, pl.ds(16, 16)] = rows[k, pl.ds(16, 16)] * sv

        for j in range(3):
            s = 3 * core + j
            for layer in range(2):
                table = (X_hbm if layer == 0 else L1).at[s]
                Ldst = (L1 if layer == 0 else L2).at[s]

                @pl.loop(0, 56)
                def _(i):
                    pltpu.sync_copy(
                        zbuf, shared.at[pl.ds(share_off + i * 55, 55)])

                @pl.when(tid < 15)
                def _():
                    pltpu.sync_copy(zbuf.at[pl.ds(0, 48)],
                                    shared.at[pl.ds(share_off + 3080, 48)])
                plsc.subcore_barrier()

                stage(tvA, hA, 0, stsA)
                gather_start(table, tvA, rowsA, gsA)
                stage(tvB, hB, 1, stsB)
                gather_start(table, tvB, rowsB, gsB)
                stage(tvC, hC, 2, stsC)
                gather_start(table, tvC, rowsC, gsC)

                @pl.loop(0, CPT // 3)
                def _(g):
                    c0 = 3 * g
                    slots = ((tvA, hA, rowsA, stsA, gsA, ssA),
                             (tvB, hB, rowsB, stsB, gsB, ssB),
                             (tvC, hC, rowsC, stsC, gsC, ssC))
                    scs = []
                    for tv, hb, rows, sts, gs, ss in slots:
                        gather_wait(table, tv, rows, gs)
                        scale(rows, tv)
                        scs.append(scatter(hb, rows, ss))
                    for k, (tv, hb, rows, sts, gs, ss) in enumerate(slots):
                        for cp in scs[k]:
                            cp.wait()
                        stage(tv, hb, c0 + 3 + k, sts)
                        gather_start(table, tv, rows, gs)

                gather_wait(table, tvA, rowsA, gsA)
                gather_wait(table, tvB, rowsB, gsB)
                gather_wait(table, tvC, rowsC, gsC)
                plsc.subcore_barrier()

                pltpu.sync_copy(shared.at[pl.ds(share_off, 3080)],
                                Ldst.at[pl.ds(share_off, 3080)])

                @pl.when(tid < 15)
                def _():
                    pltpu.sync_copy(shared.at[pl.ds(share_off + 3080, 48)],
                                    Ldst.at[pl.ds(share_off + 3080, 48)])
                plsc.subcore_barrier()

            for b in range(BCH // 2):
                cps = []
                for jj in range(2):
                    cps.append(pltpu.async_copy(
                        L1.at[s].at[ibuf.at[2 * b + jj]],
                        rowsA.at[pl.ds(jj * 128, 128)], gsA))
                    cps.append(pltpu.async_copy(
                        L2.at[s].at[ibuf.at[2 * b + jj]],
                        rowsB.at[pl.ds(jj * 128, 128)], gsB))
                for cp in cps:
                    cp.wait()
                off = BCH * 128 * tid + 256 * b
                pltpu.sync_copy(rowsA, G1.at[s].at[pl.ds(off, 256)])
                pltpu.sync_copy(rowsB, G2.at[s].at[pl.ds(off, 256)])

    return k3(X, tv, h1, idx3)


def _loss(G1, G2, c, rb=512):
    nsteps = B // rb

    def k4(g1u, g1p, g1n, g2u, g2p, g2n, c_ref, o_ref):
        i = pl.program_id(0)
        K = 1.0 / c_ref[0]
        sqrtK = jnp.sqrt(K)

        def emap(g1b, g2b, m, wgt):
            a0 = (g1b[2 * m] + g2b[2 * m]) * 0.5
            a1 = (g1b[2 * m + 1] + g2b[2 * m + 1]) * 0.5
            x0 = a0[:, 0:1]
            sq = (jnp.sum(a0 * a0, axis=1, keepdims=True)
                  + jnp.sum(a1 * a1, axis=1, keepdims=True) - x0 * x0)
            yn = jnp.maximum(jnp.sqrt(jnp.maximum(sq, 0.0)), MIN_NORM)
            theta = jnp.maximum(x0 / sqrtK, 1.0 + EPS)
            ac = jnp.log(theta + jnp.sqrt(jnp.maximum(theta * theta - 1.0,
                                                      0.0)))
            f = wgt * sqrtK * ac / yn
            col = lax.broadcasted_iota(jnp.int32, a0.shape, 1)
            e0 = jnp.where(col >= 1, a0 * f, 0.0)
            e1 = a1 * f
            return e0, e1

        pos = jnp.zeros((rb, 1), jnp.float32)
        neg = jnp.zeros((rb, 1), jnp.float32)
        for m, wgt in ((0, 1.0), (1, 0.15), (2, 0.75)):
            eu0, eu1 = emap(g1u[...], g2u[...], m, wgt)
            ep0, ep1 = emap(g1p[...], g2p[...], m, wgt)
            en0, en1 = emap(g1n[...], g2n[...], m, wgt)
            pos = pos + (jnp.sum(eu0 * ep0, axis=1, keepdims=True)
                         + jnp.sum(eu1 * ep1, axis=1, keepdims=True))
            neg = neg + (jnp.sum(eu0 * en0, axis=1, keepdims=True)
                         + jnp.sum(eu1 * en1, axis=1, keepdims=True))
        d = pos - neg
        lossv = jnp.maximum(-d, 0.0) + jnp.log1p(jnp.exp(-jnp.abs(d)))
        partial = jnp.sum(lossv, axis=(0, 1), keepdims=True) / B

        @pl.when(i == 0)
        def _():
            o_ref[...] = jnp.zeros((1, 1), jnp.float32)

        o_ref[...] += partial

    blk = rb // 128
    ublock = pl.BlockSpec((NSLAB, rb, W), lambda i: (0, i, 0))
    pblock = pl.BlockSpec((NSLAB, rb, W), lambda i: (0, i + B // rb, 0))
    nblock = pl.BlockSpec((NSLAB, rb, W), lambda i: (0, i + 2 * B // rb, 0))
    return pl.pallas_call(
        k4,
        grid=(nsteps,),
        in_specs=[ublock, pblock, nblock, ublock, pblock, nblock,
                  pl.BlockSpec(memory_space=pltpu.SMEM)],
        out_specs=pl.BlockSpec((1, 1), lambda i: (0, 0)),
        out_shape=jax.ShapeDtypeStruct((1, 1), jnp.float32),
    )(G1, G1, G1, G2, G2, G2, c.reshape(1))


def _pack_edges(h, t, v):
    cap = NTILE * CPT * CHUNK
    pad = cap - E
    NPAD = CPT_ALLOC - CPT
    fill = (jnp.arange(pad, dtype=jnp.int32) * 61) % N
    fill2 = (jnp.arange(NTILE * NPAD * CHUNK, dtype=jnp.int32) * 61) % N

    def lay(x, filler):
        x_r = jnp.concatenate([x, filler]).reshape(NTILE, CPT, CHUNK)
        return jnp.concatenate(
            [x_r, fill2.reshape(NTILE, NPAD, CHUNK).astype(x.dtype)], axis=1)

    t_r = lay(t.astype(jnp.int32), fill)
    h_r = lay(h.astype(jnp.int32), fill)
    v_r = lay(jax.lax.bitcast_convert_type(v, jnp.int32),
              jnp.zeros((pad,), jnp.int32))
    tv = jnp.concatenate([t_r, v_r], axis=2)
    return tv.reshape(2 * E_ALLOC), h_r.reshape(E_ALLOC)


def kernel(user_ids, item_pos_ids, item_neg_ids, all_h_list, all_t_list,
           A_values, user_emb_id, item_emb_id, user_emb_img, user_emb_txt,
           image_feats, text_feats, Wi, bi, Wt, bt, c):
    ie_img = _matmul_bias(image_feats, Wi, bi, 400)
    ie_txt = _matmul_bias(text_feats, Wt, bt, 400)
    ego3 = jnp.stack([
        jnp.concatenate([user_emb_id, item_emb_id], axis=0),
        jnp.concatenate([user_emb_img, ie_img], axis=0),
        jnp.concatenate([user_emb_txt, ie_txt], axis=0)])
    X = _expmap_slabs(ego3, c)

    tv, h1 = _pack_edges(all_h_list, all_t_list, A_values)
    idx_all = jnp.concatenate([user_ids.astype(jnp.int32),
                               N_USERS + item_pos_ids.astype(jnp.int32),
                               N_USERS + item_neg_ids.astype(jnp.int32)])
    idx3 = idx_all.reshape(BT // 128, 128)

    _, _, G1, G2 = _sc_spmm(X, tv, h1, idx3)
    out = _loss(G1, G2, c)
    return out.reshape(())


# speedup vs baseline: 9.0231x; 1.1322x over previous
import functools

import jax
import jax.numpy as jnp
from jax import lax
from jax.experimental import pallas as pl
from jax.experimental.pallas import tpu as pltpu
from jax.experimental.pallas import tpu_sc as plsc

N_USERS = 30000
N_ITEMS = 20000
N = N_USERS + N_ITEMS
E = 800000
D = 64
B = 4096
MIN_NORM = 1e-6
EPS = 1e-7

W = 32
NSLAB = 6
NTILE = 16
CHUNK = 256
SUB = CHUNK // 128
CPT = 201
CPT_ALLOC = 204
E_ALLOC = NTILE * CPT_ALLOC * CHUNK
ROWS_PER_TILE = N // NTILE
BT = 3 * B
BCH = BT // (NTILE * 128)


def _matmul_bias(x, w, b, block_m):
    M, K = x.shape
    _, Do = w.shape

    def mm_kernel(x_ref, w_ref, b_ref, o_ref):
        o_ref[...] = (jnp.dot(x_ref[...], w_ref[...],
                              preferred_element_type=jnp.float32)
                      + b_ref[...])

    return pl.pallas_call(
        mm_kernel,
        grid=(M // block_m,),
        in_specs=[pl.BlockSpec((block_m, K), lambda i: (i, 0)),
                  pl.BlockSpec((K, Do), lambda i: (0, 0)),
                  pl.BlockSpec((1, Do), lambda i: (0, 0))],
        out_specs=pl.BlockSpec((block_m, Do), lambda i: (i, 0)),
        out_shape=jax.ShapeDtypeStruct((M, Do), jnp.float32),
    )(x, w, b.reshape(1, Do))


def _expmap_slabs(ego3, c, block_r=1000):
    def k2(ego_ref, c_ref, o_ref):
        hf = pl.program_id(1)
        u = ego_ref[0]
        K = 1.0 / c_ref[0]
        sqrtK = jnp.sqrt(K)
        col = lax.broadcasted_iota(jnp.int32, u.shape, 1)
        x = jnp.where(col >= 1, u, 0.0)
        xn = jnp.maximum(jnp.sqrt(jnp.sum(x * x, axis=1, keepdims=True)),
                         MIN_NORM)
        theta = xn / sqrtK
        et = jnp.exp(theta)
        sinh_t = 0.5 * (et - 1.0 / et)
        rest = sqrtK * sinh_t * x / xn
        y_sqnorm = jnp.sum(rest * rest, axis=1, keepdims=True)
        first = jnp.maximum(jnp.sqrt(K + y_sqnorm), EPS)
        full = jnp.where(col >= 1, rest, first)
        o_ref[0] = jnp.where(hf == 0, full[:, :W], full[:, W:])

    return pl.pallas_call(
        k2,
        grid=(3, 2, N // block_r),
        in_specs=[pl.BlockSpec((1, block_r, D), lambda m, h, r: (m, r, 0)),
                  pl.BlockSpec(memory_space=pltpu.SMEM)],
        out_specs=pl.BlockSpec((1, block_r, W),
                               lambda m, h, r: (2 * m + h, r, 0)),
        out_shape=jax.ShapeDtypeStruct((NSLAB, N, W), jnp.float32),
    )(ego3, c.reshape(1))


def _sc_spmm(X, tv, h1, idx3):
    mesh = plsc.VectorSubcoreMesh(core_axis_name="c", subcore_axis_name="s")
    fdt = jnp.float32
    out_types = (jax.ShapeDtypeStruct((NSLAB, N, W), fdt),
                 jax.ShapeDtypeStruct((NSLAB, N, W), fdt),
                 jax.ShapeDtypeStruct((NSLAB, BT, W), fdt),
                 jax.ShapeDtypeStruct((NSLAB, BT, W), fdt))

    scratch = [
        pltpu.VMEM_SHARED((N, W), fdt),
        pltpu.VMEM((55, W), fdt),
        pltpu.VMEM((2 * CHUNK,), jnp.int32),
        pltpu.VMEM((2 * CHUNK,), jnp.int32),
        pltpu.VMEM((2 * CHUNK,), jnp.int32),
        pltpu.VMEM((CHUNK,), jnp.int32),
        pltpu.VMEM((CHUNK,), jnp.int32),
        pltpu.VMEM((CHUNK,), jnp.int32),
        pltpu.VMEM((CHUNK, W), fdt),
        pltpu.VMEM((CHUNK, W), fdt),
        pltpu.VMEM((CHUNK, W), fdt),
        pltpu.VMEM((BCH, 128), jnp.int32),
        pltpu.SemaphoreType.DMA,
        pltpu.SemaphoreType.DMA,
        pltpu.SemaphoreType.DMA,
        pltpu.SemaphoreType.DMA,
        pltpu.SemaphoreType.DMA,
        pltpu.SemaphoreType.DMA,
        pltpu.SemaphoreType.DMA,
        pltpu.SemaphoreType.DMA,
        pltpu.SemaphoreType.DMA,
    ]

    @functools.partial(pl.kernel, out_type=out_types, mesh=mesh,
                       scratch_types=scratch,
                       compiler_params=pltpu.CompilerParams(
                           use_tc_tiling_on_sc=False,
                           needs_layout_passes=False))
    def k3(X_hbm, tv_hbm, h_hbm, idx_hbm, L1, L2, G1, G2,
           shared, zbuf, tvA, tvB, tvC, hA, hB, hC, rowsA, rowsB, rowsC, ibuf,
           stsA, stsB, stsC, gsA, gsB, gsC, ssA, ssB, ssC):
        core = lax.axis_index("c")
        tid = lax.axis_index("s")

        @pl.loop(0, 55, unroll=8)
        def _(i):
            zero16 = jnp.zeros((16,), fdt)
            zbuf[i, pl.ds(0, 16)] = zero16
            zbuf[i, pl.ds(16, 16)] = zero16

        share_off = tid * 3128

        pltpu.sync_copy(idx_hbm.at[pl.ds(BCH * tid, BCH)], ibuf)

        def stage(tv_buf, h_buf, ck, sts):
            base = (tid * CPT_ALLOC + ck) * CHUNK
            cps = (pltpu.async_copy(tv_hbm.at[pl.ds(2 * base, 2 * CHUNK)],
                                    tv_buf, sts),
                   pltpu.async_copy(h_hbm.at[pl.ds(base, CHUNK)], h_buf, sts))
            for cp in cps:
                cp.wait()

        def gather_start(tbl, tv_buf, rows, gs):
            pltpu.async_copy(tbl.at[tv_buf.at[pl.ds(0, CHUNK)]], rows, gs)

        def gather_wait(tbl, tv_buf, rows, gs):
            pltpu.make_async_copy(tbl.at[tv_buf.at[pl.ds(0, CHUNK)]],
                                  rows, gs).wait()

        def scatter(h_buf, rows, ss):
            return [pltpu.async_copy(rows, shared.at[h_buf], ss, add=True)]

        def scale(rows, tv_buf):
            @pl.loop(0, CHUNK, step=16)
            def _(k0):
                v16 = plsc.bitcast(tv_buf[pl.ds(CHUNK + k0, 16)], fdt)
                for i in range(16):
                    sv = lax.broadcast(v16[i], (16,))
                    k = k0 + i
                    rows[k, pl.ds(0, 16)] = rows[k, pl.ds(0, 16)] * sv
                    rows[k, pl.ds(16, 16)] = rows[k, pl.ds(16, 16)] * sv

        for j in range(3):
            s = 3 * core + j
            for layer in range(2):
                table = (X_hbm if layer == 0 else L1).at[s]
                Ldst = (L1 if layer == 0 else L2).at[s]

                @pl.loop(0, 56)
                def _(i):
                    pltpu.async_copy(
                        zbuf, shared.at[pl.ds(share_off + i * 55, 55)], stsA)

                @pl.when(tid < 15)
                def _():
                    pltpu.async_copy(zbuf.at[pl.ds(0, 48)],
                                     shared.at[pl.ds(share_off + 3080, 48)],
                                     stsB)

                @pl.loop(0, 56)
                def _(i):
                    pltpu.make_async_copy(
                        zbuf, shared.at[pl.ds(share_off + i * 55, 55)],
                        stsA).wait()

                @pl.when(tid < 15)
                def _():
                    pltpu.make_async_copy(zbuf.at[pl.ds(0, 48)],
                                          shared.at[pl.ds(share_off + 3080,
                                                          48)], stsB).wait()
                plsc.subcore_barrier()

                stage(tvA, hA, 0, stsA)
                gather_start(table, tvA, rowsA, gsA)
                stage(tvB, hB, 1, stsB)
                gather_start(table, tvB, rowsB, gsB)
                stage(tvC, hC, 2, stsC)
                gather_start(table, tvC, rowsC, gsC)

                @pl.loop(0, CPT // 3)
                def _(g):
                    c0 = 3 * g
                    slots = ((tvA, hA, rowsA, stsA, gsA, ssA),
                             (tvB, hB, rowsB, stsB, gsB, ssB),
                             (tvC, hC, rowsC, stsC, gsC, ssC))
                    scs = []
                    for tv, hb, rows, sts, gs, ss in slots:
                        gather_wait(table, tv, rows, gs)
                        scale(rows, tv)
                        scs.append(scatter(hb, rows, ss))
                    for k, (tv, hb, rows, sts, gs, ss) in enumerate(slots):
                        for cp in scs[k]:
                            cp.wait()
                        stage(tv, hb, c0 + 3 + k, sts)
                        gather_start(table, tv, rows, gs)

                gather_wait(table, tvA, rowsA, gsA)
                gather_wait(table, tvB, rowsB, gsB)
                gather_wait(table, tvC, rowsC, gsC)
                plsc.subcore_barrier()

                pltpu.sync_copy(shared.at[pl.ds(share_off, 3080)],
                                Ldst.at[pl.ds(share_off, 3080)])

                @pl.when(tid < 15)
                def _():
                    pltpu.sync_copy(shared.at[pl.ds(share_off + 3080, 48)],
                                    Ldst.at[pl.ds(share_off + 3080, 48)])
                plsc.subcore_barrier()

            for b in range(BCH // 2):
                cps = []
                for jj in range(2):
                    cps.append(pltpu.async_copy(
                        L1.at[s].at[ibuf.at[2 * b + jj]],
                        rowsA.at[pl.ds(jj * 128, 128)], gsA))
                    cps.append(pltpu.async_copy(
                        L2.at[s].at[ibuf.at[2 * b + jj]],
                        rowsB.at[pl.ds(jj * 128, 128)], gsB))
                for cp in cps:
                    cp.wait()
                off = BCH * 128 * tid + 256 * b
                pltpu.sync_copy(rowsA, G1.at[s].at[pl.ds(off, 256)])
                pltpu.sync_copy(rowsB, G2.at[s].at[pl.ds(off, 256)])

    return k3(X, tv, h1, idx3)


def _loss(G1, G2, c, rb=512):
    nsteps = B // rb

    def k4(g1u, g1p, g1n, g2u, g2p, g2n, c_ref, o_ref):
        i = pl.program_id(0)
        K = 1.0 / c_ref[0]
        sqrtK = jnp.sqrt(K)

        def emap(g1b, g2b, m, wgt):
            a0 = (g1b[2 * m] + g2b[2 * m]) * 0.5
            a1 = (g1b[2 * m + 1] + g2b[2 * m + 1]) * 0.5
            x0 = a0[:, 0:1]
            sq = (jnp.sum(a0 * a0, axis=1, keepdims=True)
                  + jnp.sum(a1 * a1, axis=1, keepdims=True) - x0 * x0)
            yn = jnp.maximum(jnp.sqrt(jnp.maximum(sq, 0.0)), MIN_NORM)
            theta = jnp.maximum(x0 / sqrtK, 1.0 + EPS)
            ac = jnp.log(theta + jnp.sqrt(jnp.maximum(theta * theta - 1.0,
                                                      0.0)))
            f = wgt * sqrtK * ac / yn
            col = lax.broadcasted_iota(jnp.int32, a0.shape, 1)
            e0 = jnp.where(col >= 1, a0 * f, 0.0)
            e1 = a1 * f
            return e0, e1

        pos = jnp.zeros((rb, 1), jnp.float32)
        neg = jnp.zeros((rb, 1), jnp.float32)
        for m, wgt in ((0, 1.0), (1, 0.15), (2, 0.75)):
            eu0, eu1 = emap(g1u[...], g2u[...], m, wgt)
            ep0, ep1 = emap(g1p[...], g2p[...], m, wgt)
            en0, en1 = emap(g1n[...], g2n[...], m, wgt)
            pos = pos + (jnp.sum(eu0 * ep0, axis=1, keepdims=True)
                         + jnp.sum(eu1 * ep1, axis=1, keepdims=True))
            neg = neg + (jnp.sum(eu0 * en0, axis=1, keepdims=True)
                         + jnp.sum(eu1 * en1, axis=1, keepdims=True))
        d = pos - neg
        lossv = jnp.maximum(-d, 0.0) + jnp.log1p(jnp.exp(-jnp.abs(d)))
        partial = jnp.sum(lossv, axis=(0, 1), keepdims=True) / B

        @pl.when(i == 0)
        def _():
            o_ref[...] = jnp.zeros((1, 1), jnp.float32)

        o_ref[...] += partial

    blk = rb // 128
    ublock = pl.BlockSpec((NSLAB, rb, W), lambda i: (0, i, 0))
    pblock = pl.BlockSpec((NSLAB, rb, W), lambda i: (0, i + B // rb, 0))
    nblock = pl.BlockSpec((NSLAB, rb, W), lambda i: (0, i + 2 * B // rb, 0))
    return pl.pallas_call(
        k4,
        grid=(nsteps,),
        in_specs=[ublock, pblock, nblock, ublock, pblock, nblock,
                  pl.BlockSpec(memory_space=pltpu.SMEM)],
        out_specs=pl.BlockSpec((1, 1), lambda i: (0, 0)),
        out_shape=jax.ShapeDtypeStruct((1, 1), jnp.float32),
    )(G1, G1, G1, G2, G2, G2, c.reshape(1))


def _pack_edges(h, t, v):
    cap = NTILE * CPT * CHUNK
    pad = cap - E
    NPAD = CPT_ALLOC - CPT
    fill = (jnp.arange(pad, dtype=jnp.int32) * 61) % N
    fill2 = (jnp.arange(NTILE * NPAD * CHUNK, dtype=jnp.int32) * 61) % N

    def lay(x, filler):
        x_r = jnp.concatenate([x, filler]).reshape(NTILE, CPT, CHUNK)
        return jnp.concatenate(
            [x_r, fill2.reshape(NTILE, NPAD, CHUNK).astype(x.dtype)], axis=1)

    t_r = lay(t.astype(jnp.int32), fill)
    h_r = lay(h.astype(jnp.int32), fill)
    v_r = lay(jax.lax.bitcast_convert_type(v, jnp.int32),
              jnp.zeros((pad,), jnp.int32))
    tv = jnp.concatenate([t_r, v_r], axis=2)
    return tv.reshape(2 * E_ALLOC), h_r.reshape(E_ALLOC)


def kernel(user_ids, item_pos_ids, item_neg_ids, all_h_list, all_t_list,
           A_values, user_emb_id, item_emb_id, user_emb_img, user_emb_txt,
           image_feats, text_feats, Wi, bi, Wt, bt, c):
    ie_img = _matmul_bias(image_feats, Wi, bi, 400)
    ie_txt = _matmul_bias(text_feats, Wt, bt, 400)
    ego3 = jnp.stack([
        jnp.concatenate([user_emb_id, item_emb_id], axis=0),
        jnp.concatenate([user_emb_img, ie_img], axis=0),
        jnp.concatenate([user_emb_txt, ie_txt], axis=0)])
    X = _expmap_slabs(ego3, c)

    tv, h1 = _pack_edges(all_h_list, all_t_list, A_values)
    idx_all = jnp.concatenate([user_ids.astype(jnp.int32),
                               N_USERS + item_pos_ids.astype(jnp.int32),
                               N_USERS + item_neg_ids.astype(jnp.int32)])
    idx3 = idx_all.reshape(BT // 128, 128)

    _, _, G1, G2 = _sc_spmm(X, tv, h1, idx3)
    out = _loss(G1, G2, c)
    return out.reshape(())


# speedup vs baseline: 9.0306x; 1.0008x over previous
import functools

import jax
import jax.numpy as jnp
from jax import lax
from jax.experimental import pallas as pl
from jax.experimental.pallas import tpu as pltpu
from jax.experimental.pallas import tpu_sc as plsc

N_USERS = 30000
N_ITEMS = 20000
N = N_USERS + N_ITEMS
E = 800000
D = 64
B = 4096
MIN_NORM = 1e-6
EPS = 1e-7

W = 32
NSLAB = 6
NTILE = 16
CHUNK = 256
SUB = CHUNK // 128
CPT = 201
CPT_ALLOC = 204
E_ALLOC = NTILE * CPT_ALLOC * CHUNK
ROWS_PER_TILE = N // NTILE
BT = 3 * B
BCH = BT // (NTILE * 128)


def _matmul_bias(x, w, b, block_m):
    M, K = x.shape
    _, Do = w.shape

    def mm_kernel(x_ref, w_ref, b_ref, o_ref):
        o_ref[...] = (jnp.dot(x_ref[...], w_ref[...],
                              preferred_element_type=jnp.float32)
                      + b_ref[...])

    return pl.pallas_call(
        mm_kernel,
        grid=(M // block_m,),
        in_specs=[pl.BlockSpec((block_m, K), lambda i: (i, 0)),
                  pl.BlockSpec((K, Do), lambda i: (0, 0)),
                  pl.BlockSpec((1, Do), lambda i: (0, 0))],
        out_specs=pl.BlockSpec((block_m, Do), lambda i: (i, 0)),
        out_shape=jax.ShapeDtypeStruct((M, Do), jnp.float32),
    )(x, w, b.reshape(1, Do))


def _expmap_slabs(ego3, c, block_r=1000):
    nm = ego3.shape[0]
    def k2(ego_ref, c_ref, o_ref):
        hf = pl.program_id(1)
        u = ego_ref[0]
        K = 1.0 / c_ref[0]
        sqrtK = jnp.sqrt(K)
        col = lax.broadcasted_iota(jnp.int32, u.shape, 1)
        x = jnp.where(col >= 1, u, 0.0)
        xn = jnp.maximum(jnp.sqrt(jnp.sum(x * x, axis=1, keepdims=True)),
                         MIN_NORM)
        theta = xn / sqrtK
        et = jnp.exp(theta)
        sinh_t = 0.5 * (et - 1.0 / et)
        rest = sqrtK * sinh_t * x / xn
        y_sqnorm = jnp.sum(rest * rest, axis=1, keepdims=True)
        first = jnp.maximum(jnp.sqrt(K + y_sqnorm), EPS)
        full = jnp.where(col >= 1, rest, first)
        o_ref[0] = jnp.where(hf == 0, full[:, :W], full[:, W:])

    return pl.pallas_call(
        k2,
        grid=(nm, 2, N // block_r),
        in_specs=[pl.BlockSpec((1, block_r, D), lambda m, h, r: (m, r, 0)),
                  pl.BlockSpec(memory_space=pltpu.SMEM)],
        out_specs=pl.BlockSpec((1, block_r, W),
                               lambda m, h, r: (2 * m + h, r, 0)),
        out_shape=jax.ShapeDtypeStruct((2 * nm, N, W), jnp.float32),
    )(ego3, c.reshape(1))


def _sc_spmm(X, tv, h1, idx3):
    mesh = plsc.VectorSubcoreMesh(core_axis_name="c", subcore_axis_name="s")
    fdt = jnp.float32
    NS = X.shape[0]
    per_core = NS // 2
    out_types = (jax.ShapeDtypeStruct((NS, N, W), fdt),
                 jax.ShapeDtypeStruct((NS, N, W), fdt),
                 jax.ShapeDtypeStruct((NS, BT, W), fdt),
                 jax.ShapeDtypeStruct((NS, BT, W), fdt))

    scratch = [
        pltpu.VMEM_SHARED((N, W), fdt),
        pltpu.VMEM((55, W), fdt),
        pltpu.VMEM((2 * CHUNK,), jnp.int32),
        pltpu.VMEM((2 * CHUNK,), jnp.int32),
        pltpu.VMEM((2 * CHUNK,), jnp.int32),
        pltpu.VMEM((CHUNK,), jnp.int32),
        pltpu.VMEM((CHUNK,), jnp.int32),
        pltpu.VMEM((CHUNK,), jnp.int32),
        pltpu.VMEM((CHUNK, W), fdt),
        pltpu.VMEM((CHUNK, W), fdt),
        pltpu.VMEM((CHUNK, W), fdt),
        pltpu.VMEM((BCH, 128), jnp.int32),
        pltpu.SemaphoreType.DMA,
        pltpu.SemaphoreType.DMA,
        pltpu.SemaphoreType.DMA,
        pltpu.SemaphoreType.DMA,
        pltpu.SemaphoreType.DMA,
        pltpu.SemaphoreType.DMA,
        pltpu.SemaphoreType.DMA,
        pltpu.SemaphoreType.DMA,
        pltpu.SemaphoreType.DMA,
    ]

    @functools.partial(pl.kernel, out_type=out_types, mesh=mesh,
                       scratch_types=scratch,
                       compiler_params=pltpu.CompilerParams(
                           use_tc_tiling_on_sc=False,
                           needs_layout_passes=False))
    def k3(X_hbm, tv_hbm, h_hbm, idx_hbm, L1, L2, G1, G2,
           shared, zbuf, tvA, tvB, tvC, hA, hB, hC, rowsA, rowsB, rowsC, ibuf,
           stsA, stsB, stsC, gsA, gsB, gsC, ssA, ssB, ssC):
        core = lax.axis_index("c")
        tid = lax.axis_index("s")

        @pl.loop(0, 55, unroll=8)
        def _(i):
            zero16 = jnp.zeros((16,), fdt)
            zbuf[i, pl.ds(0, 16)] = zero16
            zbuf[i, pl.ds(16, 16)] = zero16

        share_off = tid * 3128

        pltpu.sync_copy(idx_hbm.at[pl.ds(BCH * tid, BCH)], ibuf)

        def stage(tv_buf, h_buf, ck, sts):
            base = (tid * CPT_ALLOC + ck) * CHUNK
            cps = (pltpu.async_copy(tv_hbm.at[pl.ds(2 * base, 2 * CHUNK)],
                                    tv_buf, sts),
                   pltpu.async_copy(h_hbm.at[pl.ds(base, CHUNK)], h_buf, sts))
            for cp in cps:
                cp.wait()

        def gather_start(tbl, tv_buf, rows, gs):
            pltpu.async_copy(tbl.at[tv_buf.at[pl.ds(0, CHUNK)]], rows, gs)

        def gather_wait(tbl, tv_buf, rows, gs):
            pltpu.make_async_copy(tbl.at[tv_buf.at[pl.ds(0, CHUNK)]],
                                  rows, gs).wait()

        def scatter(h_buf, rows, ss):
            return [pltpu.async_copy(rows, shared.at[h_buf], ss, add=True)]

        def scale(rows, tv_buf):
            @pl.loop(0, CHUNK, step=16)
            def _(k0):
                v16 = plsc.bitcast(tv_buf[pl.ds(CHUNK + k0, 16)], fdt)
                for i in range(16):
                    sv = lax.broadcast(v16[i], (16,))
                    k = k0 + i
                    rows[k, pl.ds(0, 16)] = rows[k, pl.ds(0, 16)] * sv
                    rows[k, pl.ds(16, 16)] = rows[k, pl.ds(16, 16)] * sv

        for j in range(per_core):
            s = per_core * core + j
            for layer in range(2):
                table = (X_hbm if layer == 0 else L1).at[s]
                Ldst = (L1 if layer == 0 else L2).at[s]

                @pl.loop(0, 56)
                def _(i):
                    pltpu.async_copy(
                        zbuf, shared.at[pl.ds(share_off + i * 55, 55)], stsA)

                @pl.when(tid < 15)
                def _():
                    pltpu.async_copy(zbuf.at[pl.ds(0, 48)],
                                     shared.at[pl.ds(share_off + 3080, 48)],
                                     stsB)

                @pl.loop(0, 56)
                def _(i):
                    pltpu.make_async_copy(
                        zbuf, shared.at[pl.ds(share_off + i * 55, 55)],
                        stsA).wait()

                @pl.when(tid < 15)
                def _():
                    pltpu.make_async_copy(zbuf.at[pl.ds(0, 48)],
                                          shared.at[pl.ds(share_off + 3080,
                                                          48)], stsB).wait()
                plsc.subcore_barrier()

                stage(tvA, hA, 0, stsA)
                gather_start(table, tvA, rowsA, gsA)
                stage(tvB, hB, 1, stsB)
                gather_start(table, tvB, rowsB, gsB)
                stage(tvC, hC, 2, stsC)
                gather_start(table, tvC, rowsC, gsC)

                @pl.loop(0, CPT // 3)
                def _(g):
                    c0 = 3 * g
                    slots = ((tvA, hA, rowsA, stsA, gsA, ssA),
                             (tvB, hB, rowsB, stsB, gsB, ssB),
                             (tvC, hC, rowsC, stsC, gsC, ssC))
                    scs = []
                    for tv, hb, rows, sts, gs, ss in slots:
                        gather_wait(table, tv, rows, gs)
                        scale(rows, tv)
                        scs.append(scatter(hb, rows, ss))
                    for k, (tv, hb, rows, sts, gs, ss) in enumerate(slots):
                        for cp in scs[k]:
                            cp.wait()
                        stage(tv, hb, c0 + 3 + k, sts)
                        gather_start(table, tv, rows, gs)

                gather_wait(table, tvA, rowsA, gsA)
                gather_wait(table, tvB, rowsB, gsB)
                gather_wait(table, tvC, rowsC, gsC)
                plsc.subcore_barrier()

                pltpu.sync_copy(shared.at[pl.ds(share_off, 3080)],
                                Ldst.at[pl.ds(share_off, 3080)])

                @pl.when(tid < 15)
                def _():
                    pltpu.sync_copy(shared.at[pl.ds(share_off + 3080, 48)],
                                    Ldst.at[pl.ds(share_off + 3080, 48)])
                plsc.subcore_barrier()

            for b in range(BCH // 2):
                cps = []
                for jj in range(2):
                    cps.append(pltpu.async_copy(
                        L1.at[s].at[ibuf.at[2 * b + jj]],
                        rowsA.at[pl.ds(jj * 128, 128)], gsA))
                    cps.append(pltpu.async_copy(
                        L2.at[s].at[ibuf.at[2 * b + jj]],
                        rowsB.at[pl.ds(jj * 128, 128)], gsB))
                for cp in cps:
                    cp.wait()
                off = BCH * 128 * tid + 256 * b
                pltpu.sync_copy(rowsA, G1.at[s].at[pl.ds(off, 256)])
                pltpu.sync_copy(rowsB, G2.at[s].at[pl.ds(off, 256)])

    return k3(X, tv, h1, idx3)


def _loss(G1, G2, c, rb=512):
    nsteps = B // rb

    def k4(g1u, g1p, g1n, g2u, g2p, g2n, c_ref, o_ref):
        i = pl.program_id(0)
        K = 1.0 / c_ref[0]
        sqrtK = jnp.sqrt(K)

        def emap(g1b, g2b, m, wgt):
            a0 = (g1b[2 * m] + g2b[2 * m]) * 0.5
            a1 = (g1b[2 * m + 1] + g2b[2 * m + 1]) * 0.5
            x0 = a0[:, 0:1]
            sq = (jnp.sum(a0 * a0, axis=1, keepdims=True)
                  + jnp.sum(a1 * a1, axis=1, keepdims=True) - x0 * x0)
            yn = jnp.maximum(jnp.sqrt(jnp.maximum(sq, 0.0)), MIN_NORM)
            theta = jnp.maximum(x0 / sqrtK, 1.0 + EPS)
            ac = jnp.log(theta + jnp.sqrt(jnp.maximum(theta * theta - 1.0,
                                                      0.0)))
            f = wgt * sqrtK * ac / yn
            col = lax.broadcasted_iota(jnp.int32, a0.shape, 1)
            e0 = jnp.where(col >= 1, a0 * f, 0.0)
            e1 = a1 * f
            return e0, e1

        pos = jnp.zeros((rb, 1), jnp.float32)
        neg = jnp.zeros((rb, 1), jnp.float32)
        for m, wgt in ((0, 1.0), (1, 0.15), (2, 0.75)):
            eu0, eu1 = emap(g1u[...], g2u[...], m, wgt)
            ep0, ep1 = emap(g1p[...], g2p[...], m, wgt)
            en0, en1 = emap(g1n[...], g2n[...], m, wgt)
            pos = pos + (jnp.sum(eu0 * ep0, axis=1, keepdims=True)
                         + jnp.sum(eu1 * ep1, axis=1, keepdims=True))
            neg = neg + (jnp.sum(eu0 * en0, axis=1, keepdims=True)
                         + jnp.sum(eu1 * en1, axis=1, keepdims=True))
        d = pos - neg
        lossv = jnp.maximum(-d, 0.0) + jnp.log1p(jnp.exp(-jnp.abs(d)))
        partial = jnp.sum(lossv, axis=(0, 1), keepdims=True) / B

        @pl.when(i == 0)
        def _():
            o_ref[...] = jnp.zeros((1, 1), jnp.float32)

        o_ref[...] += partial

    blk = rb // 128
    ublock = pl.BlockSpec((NSLAB, rb, W), lambda i: (0, i, 0))
    pblock = pl.BlockSpec((NSLAB, rb, W), lambda i: (0, i + B // rb, 0))
    nblock = pl.BlockSpec((NSLAB, rb, W), lambda i: (0, i + 2 * B // rb, 0))
    return pl.pallas_call(
        k4,
        grid=(nsteps,),
        in_specs=[ublock, pblock, nblock, ublock, pblock, nblock,
                  pl.BlockSpec(memory_space=pltpu.SMEM)],
        out_specs=pl.BlockSpec((1, 1), lambda i: (0, 0)),
        out_shape=jax.ShapeDtypeStruct((1, 1), jnp.float32),
    )(G1, G1, G1, G2, G2, G2, c.reshape(1))


def _pack_edges(h, t, v):
    cap = NTILE * CPT * CHUNK
    pad = cap - E
    NPAD = CPT_ALLOC - CPT
    fill = (jnp.arange(pad, dtype=jnp.int32) * 61) % N
    fill2 = (jnp.arange(NTILE * NPAD * CHUNK, dtype=jnp.int32) * 61) % N

    def lay(x, filler):
        x_r = jnp.concatenate([x, filler]).reshape(NTILE, CPT, CHUNK)
        return jnp.concatenate(
            [x_r, fill2.reshape(NTILE, NPAD, CHUNK).astype(x.dtype)], axis=1)

    t_r = lay(t.astype(jnp.int32), fill)
    h_r = lay(h.astype(jnp.int32), fill)
    v_r = lay(jax.lax.bitcast_convert_type(v, jnp.int32),
              jnp.zeros((pad,), jnp.int32))
    tv = jnp.concatenate([t_r, v_r], axis=2)
    return tv.reshape(2 * E_ALLOC), h_r.reshape(E_ALLOC)


def kernel(user_ids, item_pos_ids, item_neg_ids, all_h_list, all_t_list,
           A_values, user_emb_id, item_emb_id, user_emb_img, user_emb_txt,
           image_feats, text_feats, Wi, bi, Wt, bt, c):
    tv, h1 = _pack_edges(all_h_list, all_t_list, A_values)
    idx_all = jnp.concatenate([user_ids.astype(jnp.int32),
                               N_USERS + item_pos_ids.astype(jnp.int32),
                               N_USERS + item_neg_ids.astype(jnp.int32)])
    idx3 = idx_all.reshape(BT // 128, 128)

    ego_id = jnp.concatenate([user_emb_id, item_emb_id], axis=0)
    X0 = _expmap_slabs(ego_id[None], c)
    _, _, G1_0, G2_0 = _sc_spmm(X0, tv, h1, idx3)

    ie_txt = _matmul_bias(text_feats, Wt, bt, 400)
    ego_txt = jnp.concatenate([user_emb_txt, ie_txt], axis=0)
    X2 = _expmap_slabs(ego_txt[None], c)
    _, _, G1_2, G2_2 = _sc_spmm(X2, tv, h1, idx3)

    ie_img = _matmul_bias(image_feats, Wi, bi, 400)
    ego_img = jnp.concatenate([user_emb_img, ie_img], axis=0)
    X1 = _expmap_slabs(ego_img[None], c)
    _, _, G1_1, G2_1 = _sc_spmm(X1, tv, h1, idx3)

    G1 = jnp.concatenate([G1_0, G1_1, G1_2], axis=0)
    G2 = jnp.concatenate([G2_0, G2_1, G2_2], axis=0)
    out = _loss(G1, G2, c)
    return out.reshape(())


# speedup vs baseline: 9.2532x; 1.0246x over previous
import functools

import jax
import jax.numpy as jnp
from jax import lax
from jax.experimental import pallas as pl
from jax.experimental.pallas import tpu as pltpu
from jax.experimental.pallas import tpu_sc as plsc

N_USERS = 30000
N_ITEMS = 20000
N = N_USERS + N_ITEMS
E = 800000
D = 64
B = 4096
MIN_NORM = 1e-6
EPS = 1e-7

W = 32
NSLAB = 6
NTILE = 16
CHUNK = 256
SUB = CHUNK // 128
CPT = 201
CPT_ALLOC = 204
E_ALLOC = NTILE * CPT_ALLOC * CHUNK
ROWS_PER_TILE = N // NTILE
BT = 3 * B
BCH = BT // (NTILE * 128)


def _matmul_bias(x, w, b, block_m):
    M, K = x.shape
    _, Do = w.shape

    def mm_kernel(x_ref, w_ref, b_ref, o_ref):
        o_ref[...] = (jnp.dot(x_ref[...], w_ref[...],
                              preferred_element_type=jnp.float32)
                      + b_ref[...])

    return pl.pallas_call(
        mm_kernel,
        grid=(M // block_m,),
        in_specs=[pl.BlockSpec((block_m, K), lambda i: (i, 0)),
                  pl.BlockSpec((K, Do), lambda i: (0, 0)),
                  pl.BlockSpec((1, Do), lambda i: (0, 0))],
        out_specs=pl.BlockSpec((block_m, Do), lambda i: (i, 0)),
        out_shape=jax.ShapeDtypeStruct((M, Do), jnp.float32),
    )(x, w, b.reshape(1, Do))


def _expmap_slabs(ego3, c, block_r=1000):
    nm = ego3.shape[0]

    def k2(ego_ref, c_ref, o_ref):
        u = ego_ref[0]
        K = 1.0 / c_ref[0]
        sqrtK = jnp.sqrt(K)
        col = lax.broadcasted_iota(jnp.int32, u.shape, 1)
        x = jnp.where(col >= 1, u, 0.0)
        xn = jnp.maximum(jnp.sqrt(jnp.sum(x * x, axis=1, keepdims=True)),
                         MIN_NORM)
        theta = xn / sqrtK
        et = jnp.exp(theta)
        sinh_t = 0.5 * (et - 1.0 / et)
        rest = sqrtK * sinh_t * x / xn
        y_sqnorm = jnp.sum(rest * rest, axis=1, keepdims=True)
        first = jnp.maximum(jnp.sqrt(K + y_sqnorm), EPS)
        full = jnp.where(col >= 1, rest, first)
        o_ref[0] = full[:, :W]
        o_ref[1] = full[:, W:]

    return pl.pallas_call(
        k2,
        grid=(nm, N // block_r),
        in_specs=[pl.BlockSpec((1, block_r, D), lambda m, r: (m, r, 0)),
                  pl.BlockSpec(memory_space=pltpu.SMEM)],
        out_specs=pl.BlockSpec((2, block_r, W), lambda m, r: (2 * m, r, 0)),
        out_shape=jax.ShapeDtypeStruct((2 * nm, N, W), jnp.float32),
    )(ego3, c.reshape(1))


def _sc_spmm(X, tv, h1, idx3):
    mesh = plsc.VectorSubcoreMesh(core_axis_name="c", subcore_axis_name="s")
    fdt = jnp.float32
    NS = X.shape[0]
    per_core = NS // 2
    out_types = (jax.ShapeDtypeStruct((NS, N, W), fdt),
                 jax.ShapeDtypeStruct((NS, N, W), fdt),
                 jax.ShapeDtypeStruct((NS, BT, W), fdt),
                 jax.ShapeDtypeStruct((NS, BT, W), fdt))

    scratch = [
        pltpu.VMEM_SHARED((N, W), fdt),
        pltpu.VMEM((55, W), fdt),
        pltpu.VMEM((2 * CHUNK,), jnp.int32),
        pltpu.VMEM((2 * CHUNK,), jnp.int32),
        pltpu.VMEM((2 * CHUNK,), jnp.int32),
        pltpu.VMEM((CHUNK,), jnp.int32),
        pltpu.VMEM((CHUNK,), jnp.int32),
        pltpu.VMEM((CHUNK,), jnp.int32),
        pltpu.VMEM((CHUNK, W), fdt),
        pltpu.VMEM((CHUNK, W), fdt),
        pltpu.VMEM((CHUNK, W), fdt),
        pltpu.VMEM((BCH, 128), jnp.int32),
        pltpu.SemaphoreType.DMA,
        pltpu.SemaphoreType.DMA,
        pltpu.SemaphoreType.DMA,
        pltpu.SemaphoreType.DMA,
        pltpu.SemaphoreType.DMA,
        pltpu.SemaphoreType.DMA,
        pltpu.SemaphoreType.DMA,
        pltpu.SemaphoreType.DMA,
        pltpu.SemaphoreType.DMA,
    ]

    @functools.partial(pl.kernel, out_type=out_types, mesh=mesh,
                       scratch_types=scratch,
                       compiler_params=pltpu.CompilerParams(
                           use_tc_tiling_on_sc=False,
                           needs_layout_passes=False))
    def k3(X_hbm, tv_hbm, h_hbm, idx_hbm, L1, L2, G1, G2,
           shared, zbuf, tvA, tvB, tvC, hA, hB, hC, rowsA, rowsB, rowsC, ibuf,
           stsA, stsB, stsC, gsA, gsB, gsC, ssA, ssB, ssC):
        core = lax.axis_index("c")
        tid = lax.axis_index("s")

        @pl.loop(0, 55, unroll=8)
        def _(i):
            zero16 = jnp.zeros((16,), fdt)
            zbuf[i, pl.ds(0, 16)] = zero16
            zbuf[i, pl.ds(16, 16)] = zero16

        share_off = tid * 3128

        pltpu.sync_copy(idx_hbm.at[pl.ds(BCH * tid, BCH)], ibuf)

        def stage(tv_buf, h_buf, ck, sts):
            base = (tid * CPT_ALLOC + ck) * CHUNK
            cps = (pltpu.async_copy(tv_hbm.at[pl.ds(2 * base, 2 * CHUNK)],
                                    tv_buf, sts),
                   pltpu.async_copy(h_hbm.at[pl.ds(base, CHUNK)], h_buf, sts))
            for cp in cps:
                cp.wait()

        def gather_start(tbl, tv_buf, rows, gs):
            pltpu.async_copy(tbl.at[tv_buf.at[pl.ds(0, CHUNK)]], rows, gs)

        def gather_wait(tbl, tv_buf, rows, gs):
            pltpu.make_async_copy(tbl.at[tv_buf.at[pl.ds(0, CHUNK)]],
                                  rows, gs).wait()

        def scatter(h_buf, rows, ss):
            return [pltpu.async_copy(rows, shared.at[h_buf], ss, add=True)]

        def scale(rows, tv_buf):
            @pl.loop(0, CHUNK, step=16)
            def _(k0):
                v16 = plsc.bitcast(tv_buf[pl.ds(CHUNK + k0, 16)], fdt)
                for i in range(16):
                    sv = lax.broadcast(v16[i], (16,))
                    k = k0 + i
                    rows[k, pl.ds(0, 16)] = rows[k, pl.ds(0, 16)] * sv
                    rows[k, pl.ds(16, 16)] = rows[k, pl.ds(16, 16)] * sv

        for j in range(per_core):
            s = per_core * core + j
            for layer in range(2):
                table = (X_hbm if layer == 0 else L1).at[s]
                Ldst = (L1 if layer == 0 else L2).at[s]

                @pl.loop(0, 56)
                def _(i):
                    pltpu.async_copy(
                        zbuf, shared.at[pl.ds(share_off + i * 55, 55)], stsA)

                @pl.when(tid < 15)
                def _():
                    pltpu.async_copy(zbuf.at[pl.ds(0, 48)],
                                     shared.at[pl.ds(share_off + 3080, 48)],
                                     stsB)

                @pl.loop(0, 56)
                def _(i):
                    pltpu.make_async_copy(
                        zbuf, shared.at[pl.ds(share_off + i * 55, 55)],
                        stsA).wait()

                @pl.when(tid < 15)
                def _():
                    pltpu.make_async_copy(zbuf.at[pl.ds(0, 48)],
                                          shared.at[pl.ds(share_off + 3080,
                                                          48)], stsB).wait()
                plsc.subcore_barrier()

                stage(tvA, hA, 0, stsA)
                gather_start(table, tvA, rowsA, gsA)
                stage(tvB, hB, 1, stsB)
                gather_start(table, tvB, rowsB, gsB)
                stage(tvC, hC, 2, stsC)
                gather_start(table, tvC, rowsC, gsC)

                @pl.loop(0, CPT // 3)
                def _(g):
                    c0 = 3 * g
                    slots = ((tvA, hA, rowsA, stsA, gsA, ssA),
                             (tvB, hB, rowsB, stsB, gsB, ssB),
                             (tvC, hC, rowsC, stsC, gsC, ssC))
                    scs = []
                    for tv, hb, rows, sts, gs, ss in slots:
                        gather_wait(table, tv, rows, gs)
                        scale(rows, tv)
                        scs.append(scatter(hb, rows, ss))
                    for k, (tv, hb, rows, sts, gs, ss) in enumerate(slots):
                        for cp in scs[k]:
                            cp.wait()
                        stage(tv, hb, c0 + 3 + k, sts)
                        gather_start(table, tv, rows, gs)

                gather_wait(table, tvA, rowsA, gsA)
                gather_wait(table, tvB, rowsB, gsB)
                gather_wait(table, tvC, rowsC, gsC)
                plsc.subcore_barrier()

                pltpu.sync_copy(shared.at[pl.ds(share_off, 3080)],
                                Ldst.at[pl.ds(share_off, 3080)])

                @pl.when(tid < 15)
                def _():
                    pltpu.sync_copy(shared.at[pl.ds(share_off + 3080, 48)],
                                    Ldst.at[pl.ds(share_off + 3080, 48)])
                plsc.subcore_barrier()

            for b in range(BCH // 2):
                cps = []
                for jj in range(2):
                    cps.append(pltpu.async_copy(
                        L1.at[s].at[ibuf.at[2 * b + jj]],
                        rowsA.at[pl.ds(jj * 128, 128)], gsA))
                    cps.append(pltpu.async_copy(
                        L2.at[s].at[ibuf.at[2 * b + jj]],
                        rowsB.at[pl.ds(jj * 128, 128)], gsB))
                for cp in cps:
                    cp.wait()
                off = BCH * 128 * tid + 256 * b
                pltpu.sync_copy(rowsA, G1.at[s].at[pl.ds(off, 256)])
                pltpu.sync_copy(rowsB, G2.at[s].at[pl.ds(off, 256)])

    return k3(X, tv, h1, idx3)


def _loss(G1, G2, c, rb=512):
    nsteps = B // rb

    def k4(g1u, g1p, g1n, g2u, g2p, g2n, c_ref, o_ref):
        i = pl.program_id(0)
        K = 1.0 / c_ref[0]
        sqrtK = jnp.sqrt(K)

        def emap(g1b, g2b, m, wgt):
            a0 = (g1b[2 * m] + g2b[2 * m]) * 0.5
            a1 = (g1b[2 * m + 1] + g2b[2 * m + 1]) * 0.5
            x0 = a0[:, 0:1]
            sq = (jnp.sum(a0 * a0, axis=1, keepdims=True)
                  + jnp.sum(a1 * a1, axis=1, keepdims=True) - x0 * x0)
            yn = jnp.maximum(jnp.sqrt(jnp.maximum(sq, 0.0)), MIN_NORM)
            theta = jnp.maximum(x0 / sqrtK, 1.0 + EPS)
            ac = jnp.log(theta + jnp.sqrt(jnp.maximum(theta * theta - 1.0,
                                                      0.0)))
            f = wgt * sqrtK * ac / yn
            col = lax.broadcasted_iota(jnp.int32, a0.shape, 1)
            e0 = jnp.where(col >= 1, a0 * f, 0.0)
            e1 = a1 * f
            return e0, e1

        pos = jnp.zeros((rb, 1), jnp.float32)
        neg = jnp.zeros((rb, 1), jnp.float32)
        for m, wgt in ((0, 1.0), (1, 0.15), (2, 0.75)):
            eu0, eu1 = emap(g1u[...], g2u[...], m, wgt)
            ep0, ep1 = emap(g1p[...], g2p[...], m, wgt)
            en0, en1 = emap(g1n[...], g2n[...], m, wgt)
            pos = pos + (jnp.sum(eu0 * ep0, axis=1, keepdims=True)
                         + jnp.sum(eu1 * ep1, axis=1, keepdims=True))
            neg = neg + (jnp.sum(eu0 * en0, axis=1, keepdims=True)
                         + jnp.sum(eu1 * en1, axis=1, keepdims=True))
        d = pos - neg
        lossv = jnp.maximum(-d, 0.0) + jnp.log1p(jnp.exp(-jnp.abs(d)))
        partial = jnp.sum(lossv, axis=(0, 1), keepdims=True) / B

        @pl.when(i == 0)
        def _():
            o_ref[...] = jnp.zeros((1, 1), jnp.float32)

        o_ref[...] += partial

    blk = rb // 128
    ublock = pl.BlockSpec((NSLAB, rb, W), lambda i: (0, i, 0))
    pblock = pl.BlockSpec((NSLAB, rb, W), lambda i: (0, i + B // rb, 0))
    nblock = pl.BlockSpec((NSLAB, rb, W), lambda i: (0, i + 2 * B // rb, 0))
    return pl.pallas_call(
        k4,
        grid=(nsteps,),
        in_specs=[ublock, pblock, nblock, ublock, pblock, nblock,
                  pl.BlockSpec(memory_space=pltpu.SMEM)],
        out_specs=pl.BlockSpec((1, 1), lambda i: (0, 0)),
        out_shape=jax.ShapeDtypeStruct((1, 1), jnp.float32),
    )(G1, G1, G1, G2, G2, G2, c.reshape(1))


def _pack_edges(h, t, v):
    cap = NTILE * CPT * CHUNK
    pad = cap - E
    NPAD = CPT_ALLOC - CPT
    fill = (jnp.arange(pad, dtype=jnp.int32) * 61) % N
    fill2 = (jnp.arange(NTILE * NPAD * CHUNK, dtype=jnp.int32) * 61) % N

    def lay(x, filler):
        x_r = jnp.concatenate([x, filler]).reshape(NTILE, CPT, CHUNK)
        return jnp.concatenate(
            [x_r, fill2.reshape(NTILE, NPAD, CHUNK).astype(x.dtype)], axis=1)

    t_r = lay(t.astype(jnp.int32), fill)
    h_r = lay(h.astype(jnp.int32), fill)
    v_r = lay(jax.lax.bitcast_convert_type(v, jnp.int32),
              jnp.zeros((pad,), jnp.int32))
    tv = jnp.concatenate([t_r, v_r], axis=2)
    return tv.reshape(2 * E_ALLOC), h_r.reshape(E_ALLOC)


def kernel(user_ids, item_pos_ids, item_neg_ids, all_h_list, all_t_list,
           A_values, user_emb_id, item_emb_id, user_emb_img, user_emb_txt,
           image_feats, text_feats, Wi, bi, Wt, bt, c):
    tv, h1 = _pack_edges(all_h_list, all_t_list, A_values)
    idx_all = jnp.concatenate([user_ids.astype(jnp.int32),
                               N_USERS + item_pos_ids.astype(jnp.int32),
                               N_USERS + item_neg_ids.astype(jnp.int32)])
    idx3 = idx_all.reshape(BT // 128, 128)

    ego_id = jnp.concatenate([user_emb_id, item_emb_id], axis=0)
    X0 = _expmap_slabs(ego_id[None], c)
    _, _, G1_0, G2_0 = _sc_spmm(X0, tv, h1, idx3)

    ie_txt = _matmul_bias(text_feats, Wt, bt, 400)
    ego_txt = jnp.concatenate([user_emb_txt, ie_txt], axis=0)
    X2 = _expmap_slabs(ego_txt[None], c)
    _, _, G1_2, G2_2 = _sc_spmm(X2, tv, h1, idx3)

    ie_img = _matmul_bias(image_feats, Wi, bi, 400)
    ego_img = jnp.concatenate([user_emb_img, ie_img], axis=0)
    X1 = _expmap_slabs(ego_img[None], c)
    _, _, G1_1, G2_1 = _sc_spmm(X1, tv, h1, idx3)

    G1 = jnp.concatenate([G1_0, G1_1, G1_2], axis=0)
    G2 = jnp.concatenate([G2_0, G2_1, G2_2], axis=0)
    out = _loss(G1, G2, c)
    return out.reshape(())


# speedup vs baseline: 9.2549x; 1.0002x over previous
import functools

import jax
import jax.numpy as jnp
from jax import lax
from jax.experimental import pallas as pl
from jax.experimental.pallas import tpu as pltpu
from jax.experimental.pallas import tpu_sc as plsc

N_USERS = 30000
N_ITEMS = 20000
N = N_USERS + N_ITEMS
E = 800000
D = 64
B = 4096
MIN_NORM = 1e-6
EPS = 1e-7

W = 32
NSLAB = 6
NTILE = 16
CHUNK = 256
CPT = 201
CPT_ALLOC = 204
E_ALLOC = NTILE * CPT_ALLOC * CHUNK
BT = 3 * B
BCH = BT // (NTILE * 128)


def _matmul_bias(x, w, b, block_m):
    M, K = x.shape
    _, Do = w.shape

    def mm_kernel(x_ref, w_ref, b_ref, o_ref):
        o_ref[...] = (jnp.dot(x_ref[...], w_ref[...],
                              preferred_element_type=jnp.float32)
                      + b_ref[...])

    return pl.pallas_call(
        mm_kernel,
        grid=(M // block_m,),
        in_specs=[pl.BlockSpec((block_m, K), lambda i: (i, 0)),
                  pl.BlockSpec((K, Do), lambda i: (0, 0)),
                  pl.BlockSpec((1, Do), lambda i: (0, 0))],
        out_specs=pl.BlockSpec((block_m, Do), lambda i: (i, 0)),
        out_shape=jax.ShapeDtypeStruct((M, Do), jnp.float32),
    )(x, w, b.reshape(1, Do))


def _expmap_slabs(ego3, c, block_r=1000):
    nm = ego3.shape[0]

    def k2(ego_ref, c_ref, o_ref):
        u = ego_ref[0]
        K = 1.0 / c_ref[0]
        sqrtK = jnp.sqrt(K)
        col = lax.broadcasted_iota(jnp.int32, u.shape, 1)
        x = jnp.where(col >= 1, u, 0.0)
        xn = jnp.maximum(jnp.sqrt(jnp.sum(x * x, axis=1, keepdims=True)),
                         MIN_NORM)
        theta = xn / sqrtK
        et = jnp.exp(theta)
        sinh_t = 0.5 * (et - 1.0 / et)
        rest = sqrtK * sinh_t * x / xn
        y_sqnorm = jnp.sum(rest * rest, axis=1, keepdims=True)
        first = jnp.maximum(jnp.sqrt(K + y_sqnorm), EPS)
        full = jnp.where(col >= 1, rest, first)
        o_ref[0] = full[:, :W]
        o_ref[1] = full[:, W:]

    return pl.pallas_call(
        k2,
        grid=(nm, N // block_r),
        in_specs=[pl.BlockSpec((1, block_r, D), lambda m, r: (m, r, 0)),
                  pl.BlockSpec(memory_space=pltpu.SMEM)],
        out_specs=pl.BlockSpec((2, block_r, W), lambda m, r: (2 * m, r, 0)),
        out_shape=jax.ShapeDtypeStruct((2 * nm, N, W), jnp.float32),
    )(ego3, c.reshape(1))


def _sc_spmm(X, tv, h1, idx3):
    mesh = plsc.VectorSubcoreMesh(core_axis_name="c", subcore_axis_name="s")
    fdt = jnp.float32
    NS = X.shape[0]
    per_core = NS // 2
    out_types = (jax.ShapeDtypeStruct((NS, N, W), fdt),
                 jax.ShapeDtypeStruct((NS, N, W), fdt),
                 jax.ShapeDtypeStruct((NS, BT, W), fdt),
                 jax.ShapeDtypeStruct((NS, BT, W), fdt))

    scratch = [
        pltpu.VMEM_SHARED((N, W), fdt),
        pltpu.VMEM((55, W), fdt),
        pltpu.VMEM((2 * CHUNK,), jnp.int32),
        pltpu.VMEM((2 * CHUNK,), jnp.int32),
        pltpu.VMEM((2 * CHUNK,), jnp.int32),
        pltpu.VMEM((CHUNK,), jnp.int32),
        pltpu.VMEM((CHUNK,), jnp.int32),
        pltpu.VMEM((CHUNK,), jnp.int32),
        pltpu.VMEM((CHUNK, W), fdt),
        pltpu.VMEM((CHUNK, W), fdt),
        pltpu.VMEM((CHUNK, W), fdt),
        pltpu.VMEM((BCH, 128), jnp.int32),
        pltpu.SemaphoreType.DMA,
        pltpu.SemaphoreType.DMA,
        pltpu.SemaphoreType.DMA,
        pltpu.SemaphoreType.DMA,
        pltpu.SemaphoreType.DMA,
        pltpu.SemaphoreType.DMA,
        pltpu.SemaphoreType.DMA,
        pltpu.SemaphoreType.DMA,
        pltpu.SemaphoreType.DMA,
    ]

    @functools.partial(pl.kernel, out_type=out_types, mesh=mesh,
                       scratch_types=scratch,
                       compiler_params=pltpu.CompilerParams(
                           use_tc_tiling_on_sc=False,
                           needs_layout_passes=False))
    def k3(X_hbm, tv_hbm, h_hbm, idx_hbm, L1, L2, G1, G2,
           shared, zbuf, tvA, tvB, tvC, hA, hB, hC, rowsA, rowsB, rowsC, ibuf,
           stsA, stsB, stsC, gsA, gsB, gsC, ssA, ssB, ssC):
        core = lax.axis_index("c")
        tid = lax.axis_index("s")

        @pl.loop(0, 55, unroll=8)
        def _(i):
            zero16 = jnp.zeros((16,), fdt)
            zbuf[i, pl.ds(0, 16)] = zero16
            zbuf[i, pl.ds(16, 16)] = zero16

        share_off = tid * 3128

        pltpu.sync_copy(idx_hbm.at[pl.ds(BCH * tid, BCH)], ibuf)

        def stage(tv_buf, h_buf, ck, sts):
            base = (tid * CPT_ALLOC + ck) * CHUNK
            cps = (pltpu.async_copy(tv_hbm.at[pl.ds(2 * base, 2 * CHUNK)],
                                    tv_buf, sts),
                   pltpu.async_copy(h_hbm.at[pl.ds(base, CHUNK)], h_buf, sts))
            for cp in cps:
                cp.wait()

        def gather_start(tbl, tv_buf, rows, gs):
            pltpu.async_copy(tbl.at[tv_buf.at[pl.ds(0, CHUNK)]], rows, gs)

        def gather_wait(tbl, tv_buf, rows, gs):
            pltpu.make_async_copy(tbl.at[tv_buf.at[pl.ds(0, CHUNK)]],
                                  rows, gs).wait()

        def scatter(h_buf, rows, ss):
            return [pltpu.async_copy(rows, shared.at[h_buf], ss, add=True)]

        def scale(rows, tv_buf):
            @pl.loop(0, CHUNK, step=16)
            def _(k0):
                v16 = plsc.bitcast(tv_buf[pl.ds(CHUNK + k0, 16)], fdt)
                for i in range(16):
                    sv = lax.broadcast(v16[i], (16,))
                    k = k0 + i
                    rows[k, pl.ds(0, 16)] = rows[k, pl.ds(0, 16)] * sv
                    rows[k, pl.ds(16, 16)] = rows[k, pl.ds(16, 16)] * sv

        for j in range(per_core):
            s = per_core * core + j
            for layer in range(2):
                table = (X_hbm if layer == 0 else L1).at[s]
                Ldst = (L1 if layer == 0 else L2).at[s]

                @pl.loop(0, 56)
                def _(i):
                    pltpu.async_copy(
                        zbuf, shared.at[pl.ds(share_off + i * 55, 55)], stsA)

                @pl.when(tid < 15)
                def _():
                    pltpu.async_copy(zbuf.at[pl.ds(0, 48)],
                                     shared.at[pl.ds(share_off + 3080, 48)],
                                     stsB)

                @pl.loop(0, 56)
                def _(i):
                    pltpu.make_async_copy(
                        zbuf, shared.at[pl.ds(share_off + i * 55, 55)],
                        stsA).wait()

                @pl.when(tid < 15)
                def _():
                    pltpu.make_async_copy(zbuf.at[pl.ds(0, 48)],
                                          shared.at[pl.ds(share_off + 3080,
                                                          48)], stsB).wait()
                plsc.subcore_barrier()

                stage(tvA, hA, 0, stsA)
                gather_start(table, tvA, rowsA, gsA)
                stage(tvB, hB, 1, stsB)
                gather_start(table, tvB, rowsB, gsB)
                stage(tvC, hC, 2, stsC)
                gather_start(table, tvC, rowsC, gsC)

                @pl.loop(0, CPT // 3)
                def _(g):
                    c0 = 3 * g
                    slots = ((tvA, hA, rowsA, stsA, gsA, ssA),
                             (tvB, hB, rowsB, stsB, gsB, ssB),
                             (tvC, hC, rowsC, stsC, gsC, ssC))
                    scs = []
                    for tv, hb, rows, sts, gs, ss in slots:
                        gather_wait(table, tv, rows, gs)
                        scale(rows, tv)
                        scs.append(scatter(hb, rows, ss))
                    for k, (tv, hb, rows, sts, gs, ss) in enumerate(slots):
                        for cp in scs[k]:
                            cp.wait()
                        stage(tv, hb, c0 + 3 + k, sts)
                        gather_start(table, tv, rows, gs)

                gather_wait(table, tvA, rowsA, gsA)
                gather_wait(table, tvB, rowsB, gsB)
                gather_wait(table, tvC, rowsC, gsC)
                plsc.subcore_barrier()

                pltpu.sync_copy(shared.at[pl.ds(share_off, 3080)],
                                Ldst.at[pl.ds(share_off, 3080)])

                @pl.when(tid < 15)
                def _():
                    pltpu.sync_copy(shared.at[pl.ds(share_off + 3080, 48)],
                                    Ldst.at[pl.ds(share_off + 3080, 48)])
                plsc.subcore_barrier()

            for b in range(BCH // 2):
                cps = []
                for jj in range(2):
                    cps.append(pltpu.async_copy(
                        L1.at[s].at[ibuf.at[2 * b + jj]],
                        rowsA.at[pl.ds(jj * 128, 128)], gsA))
                    cps.append(pltpu.async_copy(
                        L2.at[s].at[ibuf.at[2 * b + jj]],
                        rowsB.at[pl.ds(jj * 128, 128)], gsB))
                for cp in cps:
                    cp.wait()
                off = BCH * 128 * tid + 256 * b
                pltpu.sync_copy(rowsA, G1.at[s].at[pl.ds(off, 256)])
                pltpu.sync_copy(rowsB, G2.at[s].at[pl.ds(off, 256)])

    return k3(X, tv, h1, idx3)


def _loss(G1, G2, c, rb=512):
    nsteps = B // rb

    def k4(g1u, g1p, g1n, g2u, g2p, g2n, c_ref, o_ref):
        i = pl.program_id(0)
        K = 1.0 / c_ref[0]
        sqrtK = jnp.sqrt(K)

        def emap(g1b, g2b, m, wgt):
            a0 = (g1b[2 * m] + g2b[2 * m]) * 0.5
            a1 = (g1b[2 * m + 1] + g2b[2 * m + 1]) * 0.5
            x0 = a0[:, 0:1]
            sq = (jnp.sum(a0 * a0, axis=1, keepdims=True)
                  + jnp.sum(a1 * a1, axis=1, keepdims=True) - x0 * x0)
            yn = jnp.maximum(jnp.sqrt(jnp.maximum(sq, 0.0)), MIN_NORM)
            theta = jnp.maximum(x0 / sqrtK, 1.0 + EPS)
            ac = jnp.log(theta + jnp.sqrt(jnp.maximum(theta * theta - 1.0,
                                                      0.0)))
            f = wgt * sqrtK * ac / yn
            col = lax.broadcasted_iota(jnp.int32, a0.shape, 1)
            e0 = jnp.where(col >= 1, a0 * f, 0.0)
            e1 = a1 * f
            return e0, e1

        pos = jnp.zeros((rb, 1), jnp.float32)
        neg = jnp.zeros((rb, 1), jnp.float32)
        for m, wgt in ((0, 1.0), (1, 0.15), (2, 0.75)):
            eu0, eu1 = emap(g1u[...], g2u[...], m, wgt)
            ep0, ep1 = emap(g1p[...], g2p[...], m, wgt)
            en0, en1 = emap(g1n[...], g2n[...], m, wgt)
            pos = pos + (jnp.sum(eu0 * ep0, axis=1, keepdims=True)
                         + jnp.sum(eu1 * ep1, axis=1, keepdims=True))
            neg = neg + (jnp.sum(eu0 * en0, axis=1, keepdims=True)
                         + jnp.sum(eu1 * en1, axis=1, keepdims=True))
        d = pos - neg
        lossv = jnp.maximum(-d, 0.0) + jnp.log1p(jnp.exp(-jnp.abs(d)))
        partial = jnp.sum(lossv, axis=(0, 1), keepdims=True) / B

        @pl.when(i == 0)
        def _():
            o_ref[...] = jnp.zeros((1, 1), jnp.float32)

        o_ref[...] += partial

    blk = rb // 128
    ublock = pl.BlockSpec((NSLAB, rb, W), lambda i: (0, i, 0))
    pblock = pl.BlockSpec((NSLAB, rb, W), lambda i: (0, i + B // rb, 0))
    nblock = pl.BlockSpec((NSLAB, rb, W), lambda i: (0, i + 2 * B // rb, 0))
    return pl.pallas_call(
        k4,
        grid=(nsteps,),
        in_specs=[ublock, pblock, nblock, ublock, pblock, nblock,
                  pl.BlockSpec(memory_space=pltpu.SMEM)],
        out_specs=pl.BlockSpec((1, 1), lambda i: (0, 0)),
        out_shape=jax.ShapeDtypeStruct((1, 1), jnp.float32),
    )(G1, G1, G1, G2, G2, G2, c.reshape(1))


def _pack_edges(h, t, v):
    cap = NTILE * CPT * CHUNK
    pad = cap - E
    NPAD = CPT_ALLOC - CPT
    fill = (jnp.arange(pad, dtype=jnp.int32) * 61) % N
    fill2 = (jnp.arange(NTILE * NPAD * CHUNK, dtype=jnp.int32) * 61) % N

    def lay(x, filler):
        x_r = jnp.concatenate([x, filler]).reshape(NTILE, CPT, CHUNK)
        return jnp.concatenate(
            [x_r, fill2.reshape(NTILE, NPAD, CHUNK).astype(x.dtype)], axis=1)

    t_r = lay(t.astype(jnp.int32), fill)
    h_r = lay(h.astype(jnp.int32), fill)
    v_r = lay(jax.lax.bitcast_convert_type(v, jnp.int32),
              jnp.zeros((pad,), jnp.int32))
    tv = jnp.concatenate([t_r, v_r], axis=2)
    return tv.reshape(2 * E_ALLOC), h_r.reshape(E_ALLOC)


def kernel(user_ids, item_pos_ids, item_neg_ids, all_h_list, all_t_list,
           A_values, user_emb_id, item_emb_id, user_emb_img, user_emb_txt,
           image_feats, text_feats, Wi, bi, Wt, bt, c):
    tv, h1 = _pack_edges(all_h_list, all_t_list, A_values)
    idx_all = jnp.concatenate([user_ids.astype(jnp.int32),
                               N_USERS + item_pos_ids.astype(jnp.int32),
                               N_USERS + item_neg_ids.astype(jnp.int32)])
    idx3 = idx_all.reshape(BT // 128, 128)

    ego_id = jnp.concatenate([user_emb_id, item_emb_id], axis=0)
    X0 = _expmap_slabs(ego_id[None], c)
    _, _, G1_0, G2_0 = _sc_spmm(X0, tv, h1, idx3)

    ie_txt = _matmul_bias(text_feats, Wt, bt, 400)
    ego_txt = jnp.concatenate([user_emb_txt, ie_txt], axis=0)
    X2 = _expmap_slabs(ego_txt[None], c)
    _, _, G1_2, G2_2 = _sc_spmm(X2, tv, h1, idx3)

    ie_img = _matmul_bias(image_feats, Wi, bi, 400)
    ego_img = jnp.concatenate([user_emb_img, ie_img], axis=0)
    X1 = _expmap_slabs(ego_img[None], c)
    _, _, G1_1, G2_1 = _sc_spmm(X1, tv, h1, idx3)

    G1 = jnp.concatenate([G1_0, G1_1, G1_2], axis=0)
    G2 = jnp.concatenate([G2_0, G2_1, G2_2], axis=0)
    out = _loss(G1, G2, c)
    return out.reshape(())
